# baseline (speedup 1.0000x reference)
import functools

import jax
import jax.numpy as jnp
from jax import lax
from jax.experimental import pallas as pl
from jax.experimental.pallas import tpu as pltpu
from jax.experimental.pallas import tpu_sc as plsc

B = 16384
D = 64
NC = 2
NS = 16
L = 16
NW = NC * NS
BPW = B // NW
GROUPS = BPW // L


def _score_body(s_idx_hbm, r_idx_hbm, o_idx_hbm, ent_hbm, rel_hbm, out_hbm,
                s_idx_v, r_idx_v, o_idx_v, s_rows, r_rows, o_rows,
                scores_v, sem):
    wid = lax.axis_index("s") * NC + lax.axis_index("c")
    base = wid * BPW

    pltpu.sync_copy(s_idx_hbm.at[pl.ds(base, BPW)], s_idx_v)
    pltpu.sync_copy(r_idx_hbm.at[pl.ds(base, BPW)], r_idx_v)
    pltpu.sync_copy(o_idx_hbm.at[pl.ds(base, BPW)], o_idx_v)

    cs = pltpu.async_copy(ent_hbm.at[s_idx_v], s_rows, sem)
    cr = pltpu.async_copy(rel_hbm.at[r_idx_v], r_rows, sem)
    co = pltpu.async_copy(ent_hbm.at[o_idx_v], o_rows, sem)
    cs.wait()
    cr.wait()
    co.wait()

    iota = lax.iota(jnp.int32, L)
    perms = [(iota ^ (1 << k)).reshape(L, 1) for k in range(4)]
    dnums = lax.GatherDimensionNumbers(
        offset_dims=(), collapsed_slice_dims=(0,), start_index_map=(0,))

    def lane_sum(v):
        for p in perms:
            v = v + lax.gather(v, p, dnums, slice_sizes=(1,),
                               mode=lax.GatherScatterMode.PROMISE_IN_BOUNDS)
        return v

    zero = jnp.zeros((L,), jnp.float32)
    for g in range(GROUPS):

        def rbody(j, acc, g=g):
            b = g * L + j
            sp = (s_rows[b, pl.ds(0, L)] + s_rows[b, pl.ds(L, L)]
                  + s_rows[b, pl.ds(2 * L, L)] + s_rows[b, pl.ds(3 * L, L)])
            qp = (r_rows[b, pl.ds(0, L)] * o_rows[b, pl.ds(0, L)]
                  + r_rows[b, pl.ds(L, L)] * o_rows[b, pl.ds(L, L)]
                  + r_rows[b, pl.ds(2 * L, L)] * o_rows[b, pl.ds(2 * L, L)]
                  + r_rows[b, pl.ds(3 * L, L)] * o_rows[b, pl.ds(3 * L, L)])
            score = lane_sum(sp) * lane_sum(qp)
            return jnp.where(iota == j, score, acc)

        acc = lax.fori_loop(0, L, rbody, zero)
        scores_v[pl.ds(g * L, L)] = acc

    pltpu.sync_copy(scores_v, out_hbm.at[pl.ds(base, BPW)])


@jax.jit
def kernel(subjects, relations, objects, entity_table, relation_table):
    s = subjects.reshape(-1).astype(jnp.int32)
    r = relations.reshape(-1).astype(jnp.int32)
    o = objects.reshape(-1).astype(jnp.int32)
    mesh = plsc.VectorSubcoreMesh(core_axis_name="c", subcore_axis_name="s")
    run = functools.partial(
        pl.kernel,
        mesh=mesh,
        compiler_params=pltpu.CompilerParams(use_tc_tiling_on_sc=False),
        out_type=jax.ShapeDtypeStruct((B,), jnp.float32),
        scratch_types=[
            pltpu.VMEM((BPW,), jnp.int32),
            pltpu.VMEM((BPW,), jnp.int32),
            pltpu.VMEM((BPW,), jnp.int32),
            pltpu.VMEM((BPW, D), jnp.float32),
            pltpu.VMEM((BPW, D), jnp.float32),
            pltpu.VMEM((BPW, D), jnp.float32),
            pltpu.VMEM((BPW,), jnp.float32),
            pltpu.SemaphoreType.DMA,
        ],
    )(_score_body)
    scores = run(s, r, o, entity_table, relation_table)
    return scores.reshape(B, 1)


# speedup vs baseline: 1.5643x; 1.5643x over previous
import functools

import jax
import jax.numpy as jnp
from jax import lax
from jax.experimental import pallas as pl
from jax.experimental.pallas import tpu as pltpu
from jax.experimental.pallas import tpu_sc as plsc

B = 16384
D = 64
N_ENT = 1000000
N_REL = 1000
L = 16
NC = 2
NS = 16
NW = NC * NS
OWN = 31232
WBUF = 31744
TAIL0 = (NW - 1) * OWN + WBUF
NTAIL = N_ENT - TAIL0
CAP = 1024
OUTN = B + NW * CAP
CHUNK = 4096


def _score_body(s_idx, r_idx, o_idx, ent_t, rel_flat, tail_flat,
                out_s, out_ro,
                idx_e, idx_r, s_e, s_b, o_e, o_b, o_r, b2,
                rowbuf, relbuf, tailbuf, acc_s, acc_ro, sem):
    wid = lax.axis_index("s") * NC + lax.axis_index("c")
    lo = wid * OWN
    hi = jnp.where(wid == NW - 1, N_ENT, lo + OWN)
    base = lo
    iota = lax.iota(jnp.int32, L)
    zero_i = jnp.zeros((L,), jnp.int32)
    zero_f = jnp.zeros((L,), jnp.float32)

    cp_rel = pltpu.async_copy(rel_flat, relbuf, sem)
    cp_tail = pltpu.async_copy(tail_flat, tailbuf, sem)

    pad0 = B + wid * CAP
    for v in range(CAP // L):
        s_e[pl.ds(v * L, L)] = zero_i
        o_e[pl.ds(v * L, L)] = zero_i
        o_r[pl.ds(v * L, L)] = zero_i
        acc_s[pl.ds(v * L, L)] = zero_f
        acc_ro[pl.ds(v * L, L)] = zero_f
        pad = pad0 + v * L + iota
        s_b[pl.ds(v * L, L)] = pad
        o_b[pl.ds(v * L, L)] = pad

    cnt_s = jnp.int32(0)
    for c in range(B // CHUNK):
        pltpu.sync_copy(s_idx.at[pl.ds(c * CHUNK, CHUNK)], idx_e)

        def svec(v, cnt, c=c):
            e = idx_e[pl.ds(v * L, L)]
            m = (e >= lo) & (e < hi)
            n = plsc.all_reduce_population_count(m)[0]
            plsc.store_compressed(s_e.at[pl.ds(cnt, L)], e - base, mask=m)
            bvec = c * CHUNK + v * L + iota
            plsc.store_compressed(s_b.at[pl.ds(cnt, L)], bvec, mask=m)
            return cnt + n

        cnt_s = lax.fori_loop(0, CHUNK // L, svec, cnt_s)

    cnt_o = jnp.int32(0)
    for c in range(B // CHUNK):
        pltpu.sync_copy(o_idx.at[pl.ds(c * CHUNK, CHUNK)], idx_e)
        pltpu.sync_copy(r_idx.at[pl.ds(c * CHUNK, CHUNK)], idx_r)

        def ovec(v, cnt, c=c):
            e = idx_e[pl.ds(v * L, L)]
            m = (e >= lo) & (e < hi)
            n = plsc.all_reduce_population_count(m)[0]
            plsc.store_compressed(o_e.at[pl.ds(cnt, L)], e - base, mask=m)
            bvec = c * CHUNK + v * L + iota
            plsc.store_compressed(o_b.at[pl.ds(cnt, L)], bvec, mask=m)
            rho = idx_r[pl.ds(v * L, L)]
            plsc.store_compressed(o_r.at[pl.ds(cnt, L)], rho, mask=m)
            return cnt + n

        cnt_o = lax.fori_loop(0, CHUNK // L, ovec, cnt_o)

    nsv = (cnt_s + L - 1) >> 4
    nov = (cnt_o + L - 1) >> 4
    cp_rel.wait()
    cp_tail.wait()

    def dbody(d, carry):
        pltpu.async_copy(ent_t.at[d, pl.ds(base, WBUF)], rowbuf_main,
                         sem).wait()
        for t in range(NTAIL // L):
            rowbuf[pl.ds(WBUF + t * L, L)] = (
                tailbuf[pl.ds(d * 128 + t * L, L)])
        rel_base = d * N_REL

        def sv(v, carry2):
            el = s_e[pl.ds(v * L, L)]
            plsc.addupdate(acc_s.at[pl.ds(v * L, L)],
                           plsc.load_gather(rowbuf, [el]))
            return carry2

        lax.fori_loop(0, nsv, sv, jnp.int32(0))

        def ov(v, carry2):
            el = o_e[pl.ds(v * L, L)]
            rho = o_r[pl.ds(v * L, L)] + rel_base
            plsc.addupdate(acc_ro.at[pl.ds(v * L, L)],
                           plsc.load_gather(rowbuf, [el])
                           * plsc.load_gather(relbuf, [rho]))
            return carry2

        lax.fori_loop(0, nov, ov, jnp.int32(0))
        return carry

    rowbuf_main = rowbuf.at[pl.ds(0, WBUF)]
    lax.fori_loop(0, D, dbody, jnp.int32(0))

    for k in range(CAP // 128):
        for t in range(128 // L):
            b2[k, pl.ds(t * L, L)] = s_b[pl.ds(k * 128 + t * L, L)]
            b2[k + CAP // 128, pl.ds(t * L, L)] = (
                o_b[pl.ds(k * 128 + t * L, L)])
    waits = []
    for k in range(CAP // 128):
        waits.append(pltpu.async_copy(
            acc_s.at[pl.ds(k * 128, 128)], out_s.at[b2.at[k]], sem))
        waits.append(pltpu.async_copy(
            acc_ro.at[pl.ds(k * 128, 128)], out_ro.at[b2.at[k + CAP // 128]],
            sem))
    for w in waits:
        w.wait()


def _mul_body(a_ref, b_ref, o_ref):
    o_ref[...] = a_ref[...] * b_ref[...]


@jax.jit
def kernel(subjects, relations, objects, entity_table, relation_table):
    s = subjects.reshape(-1).astype(jnp.int32)
    r = relations.reshape(-1).astype(jnp.int32)
    o = objects.reshape(-1).astype(jnp.int32)
    ent_t = entity_table.T
    rel_flat = relation_table.T.reshape(-1)
    tail_flat = jnp.pad(entity_table[TAIL0:].T,
                        ((0, 0), (0, 128 - NTAIL))).reshape(-1)
    mesh = plsc.VectorSubcoreMesh(core_axis_name="c", subcore_axis_name="s")
    run = functools.partial(
        pl.kernel,
        mesh=mesh,
        compiler_params=pltpu.CompilerParams(needs_layout_passes=False),
        out_type=(jax.ShapeDtypeStruct((OUTN,), jnp.float32),
                  jax.ShapeDtypeStruct((OUTN,), jnp.float32)),
        scratch_types=[
            pltpu.VMEM((CHUNK,), jnp.int32),
            pltpu.VMEM((CHUNK,), jnp.int32),
            pltpu.VMEM((CAP,), jnp.int32),
            pltpu.VMEM((CAP,), jnp.int32),
            pltpu.VMEM((CAP,), jnp.int32),
            pltpu.VMEM((CAP,), jnp.int32),
            pltpu.VMEM((CAP,), jnp.int32),
            pltpu.VMEM((2 * CAP // 128, 128), jnp.int32),
            pltpu.VMEM((WBUF + NTAIL,), jnp.float32),
            pltpu.VMEM((D * N_REL,), jnp.float32),
            pltpu.VMEM((D * 128,), jnp.float32),
            pltpu.VMEM((CAP,), jnp.float32),
            pltpu.VMEM((CAP,), jnp.float32),
            pltpu.SemaphoreType.DMA,
        ],
    )(_score_body)
    out_s, out_ro = run(s, r, o, ent_t, rel_flat, tail_flat)
    fa = out_s[:B].reshape(128, 128)
    fb = out_ro[:B].reshape(128, 128)
    scores = pl.pallas_call(
        _mul_body,
        out_shape=jax.ShapeDtypeStruct((128, 128), jnp.float32),
    )(fa, fb)
    return scores.reshape(B, 1)


# speedup vs baseline: 1.7799x; 1.1378x over previous
import functools

import jax
import jax.numpy as jnp
from jax import lax
from jax.experimental import pallas as pl
from jax.experimental.pallas import tpu as pltpu
from jax.experimental.pallas import tpu_sc as plsc

B = 16384
D = 64
N_ENT = 1000000
N_REL = 1000
L = 16
NC = 2
NS = 16
NW = NC * NS
OWN = 31232
WBUF = 31744
TAIL0 = (NW - 1) * OWN + WBUF
NTAIL = N_ENT - TAIL0
CAP = 1024
OUTN = B + NW * CAP
CHUNK = 4096


def _score_body(s_idx, r_idx, o_idx, ent_t, rel_flat, tail_flat,
                out_s, out_ro,
                idx_e, idx_r, s_e, s_b, o_e, o_b, o_r, b2,
                rowbuf_a, rowbuf_b, relrow_a, relrow_b, tailbuf,
                acc_s, acc_ro, sem, sem_a, sem_b):
    wid = lax.axis_index("s") * NC + lax.axis_index("c")
    lo = wid * OWN
    hi = jnp.where(wid == NW - 1, N_ENT, lo + OWN)
    base = lo
    iota = lax.iota(jnp.int32, L)
    zero_i = jnp.zeros((L,), jnp.int32)
    zero_f = jnp.zeros((L,), jnp.float32)

    cp_tail = pltpu.async_copy(tail_flat, tailbuf, sem)

    pad0 = B + wid * CAP
    for v in range(CAP // L):
        s_e[pl.ds(v * L, L)] = zero_i
        o_e[pl.ds(v * L, L)] = zero_i
        o_r[pl.ds(v * L, L)] = zero_i
        acc_s[pl.ds(v * L, L)] = zero_f
        acc_ro[pl.ds(v * L, L)] = zero_f
        pad = pad0 + v * L + iota
        s_b[pl.ds(v * L, L)] = pad
        o_b[pl.ds(v * L, L)] = pad

    cnt_s = jnp.int32(0)
    for c in range(B // CHUNK):
        pltpu.sync_copy(s_idx.at[pl.ds(c * CHUNK, CHUNK)], idx_e)

        def svec(v, cnt, c=c):
            e = idx_e[pl.ds(v * L, L)]
            m = (e >= lo) & (e < hi)
            n = plsc.all_reduce_population_count(m)[0]
            plsc.store_compressed(s_e.at[pl.ds(cnt, L)], e - base, mask=m)
            bvec = c * CHUNK + v * L + iota
            plsc.store_compressed(s_b.at[pl.ds(cnt, L)], bvec, mask=m)
            return cnt + n

        cnt_s = lax.fori_loop(0, CHUNK // L, svec, cnt_s)

    cnt_o = jnp.int32(0)
    for c in range(B // CHUNK):
        pltpu.sync_copy(o_idx.at[pl.ds(c * CHUNK, CHUNK)], idx_e)
        pltpu.sync_copy(r_idx.at[pl.ds(c * CHUNK, CHUNK)], idx_r)

        def ovec(v, cnt, c=c):
            e = idx_e[pl.ds(v * L, L)]
            m = (e >= lo) & (e < hi)
            n = plsc.all_reduce_population_count(m)[0]
            plsc.store_compressed(o_e.at[pl.ds(cnt, L)], e - base, mask=m)
            bvec = c * CHUNK + v * L + iota
            plsc.store_compressed(o_b.at[pl.ds(cnt, L)], bvec, mask=m)
            rho = idx_r[pl.ds(v * L, L)]
            plsc.store_compressed(o_r.at[pl.ds(cnt, L)], rho, mask=m)
            return cnt + n

        cnt_o = lax.fori_loop(0, CHUNK // L, ovec, cnt_o)

    nsv = (cnt_s + L - 1) >> 4
    nov = (cnt_o + L - 1) >> 4
    cp_tail.wait()

    rbufs = (rowbuf_a, rowbuf_b)
    rb_main = (rowbuf_a.at[pl.ds(0, WBUF)], rowbuf_b.at[pl.ds(0, WBUF)])
    relrows = (relrow_a, relrow_b)
    sems = (sem_a, sem_b)
    handles = {0: pltpu.async_copy(ent_t.at[0, pl.ds(base, WBUF)],
                                   rb_main[0], sems[0])}
    rhandles = {0: pltpu.async_copy(rel_flat.at[pl.ds(0, 1024)],
                                    relrow_a, sems[0])}
    for d in range(D):
        if d + 1 < D:
            handles[d + 1] = pltpu.async_copy(
                ent_t.at[d + 1, pl.ds(base, WBUF)],
                rb_main[(d + 1) % 2], sems[(d + 1) % 2])
            rhandles[d + 1] = pltpu.async_copy(
                rel_flat.at[pl.ds((d + 1) * 1024, 1024)],
                relrows[(d + 1) % 2], sems[(d + 1) % 2])
        handles[d].wait()
        rhandles[d].wait()
        buf = rbufs[d % 2]
        for t in range(NTAIL // L):
            buf[pl.ds(WBUF + t * L, L)] = (
                tailbuf[pl.ds(d * 128 + t * L, L)])
        relrow = relrows[d % 2]

        def sv(v, carry2, buf=buf):
            el = s_e[pl.ds(v * L, L)]
            plsc.addupdate(acc_s.at[pl.ds(v * L, L)],
                           plsc.load_gather(buf, [el]))
            return carry2

        lax.fori_loop(0, nsv, sv, jnp.int32(0))

        def ov(v, carry2, buf=buf, relrow=relrow):
            el = o_e[pl.ds(v * L, L)]
            rho = o_r[pl.ds(v * L, L)]
            plsc.addupdate(acc_ro.at[pl.ds(v * L, L)],
                           plsc.load_gather(buf, [el])
                           * plsc.load_gather(relrow, [rho]))
            return carry2

        lax.fori_loop(0, nov, ov, jnp.int32(0))

    for k in range(CAP // 128):
        for t in range(128 // L):
            b2[k, pl.ds(t * L, L)] = s_b[pl.ds(k * 128 + t * L, L)]
            b2[k + CAP // 128, pl.ds(t * L, L)] = (
                o_b[pl.ds(k * 128 + t * L, L)])
    waits = []
    for k in range(CAP // 128):
        waits.append(pltpu.async_copy(
            acc_s.at[pl.ds(k * 128, 128)], out_s.at[b2.at[k]], sem))
        waits.append(pltpu.async_copy(
            acc_ro.at[pl.ds(k * 128, 128)], out_ro.at[b2.at[k + CAP // 128]],
            sem))
    for w in waits:
        w.wait()


def _mul_body(a_ref, b_ref, o_ref):
    o_ref[...] = a_ref[...] * b_ref[...]


@jax.jit
def kernel(subjects, relations, objects, entity_table, relation_table):
    s = subjects.reshape(-1).astype(jnp.int32)
    r = relations.reshape(-1).astype(jnp.int32)
    o = objects.reshape(-1).astype(jnp.int32)
    ent_t = entity_table.T
    rel_flat = jnp.pad(relation_table.T,
                       ((0, 0), (0, 1024 - N_REL))).reshape(-1)
    tail_flat = jnp.pad(entity_table[TAIL0:].T,
                        ((0, 0), (0, 128 - NTAIL))).reshape(-1)
    mesh = plsc.VectorSubcoreMesh(core_axis_name="c", subcore_axis_name="s")
    run = functools.partial(
        pl.kernel,
        mesh=mesh,
        compiler_params=pltpu.CompilerParams(needs_layout_passes=False),
        out_type=(jax.ShapeDtypeStruct((OUTN,), jnp.float32),
                  jax.ShapeDtypeStruct((OUTN,), jnp.float32)),
        scratch_types=[
            pltpu.VMEM((CHUNK,), jnp.int32),
            pltpu.VMEM((CHUNK,), jnp.int32),
            pltpu.VMEM((CAP,), jnp.int32),
            pltpu.VMEM((CAP,), jnp.int32),
            pltpu.VMEM((CAP,), jnp.int32),
            pltpu.VMEM((CAP,), jnp.int32),
            pltpu.VMEM((CAP,), jnp.int32),
            pltpu.VMEM((2 * CAP // 128, 128), jnp.int32),
            pltpu.VMEM((WBUF + NTAIL,), jnp.float32),
            pltpu.VMEM((WBUF + NTAIL,), jnp.float32),
            pltpu.VMEM((1024,), jnp.float32),
            pltpu.VMEM((1024,), jnp.float32),
            pltpu.VMEM((D * 128,), jnp.float32),
            pltpu.VMEM((CAP,), jnp.float32),
            pltpu.VMEM((CAP,), jnp.float32),
            pltpu.SemaphoreType.DMA,
            pltpu.SemaphoreType.DMA,
            pltpu.SemaphoreType.DMA,
        ],
    )(_score_body)
    out_s, out_ro = run(s, r, o, ent_t, rel_flat, tail_flat)
    fa = out_s[:B].reshape(128, 128)
    fb = out_ro[:B].reshape(128, 128)
    scores = pl.pallas_call(
        _mul_body,
        out_shape=jax.ShapeDtypeStruct((128, 128), jnp.float32),
    )(fa, fb)
    return scores.reshape(B, 1)


# speedup vs baseline: 2.9961x; 1.6833x over previous
import functools

import jax
import jax.numpy as jnp
from jax import lax
from jax.experimental import pallas as pl
from jax.experimental.pallas import tpu as pltpu
from jax.experimental.pallas import tpu_sc as plsc

B = 16384
D = 64
N_ENT = 1000000
N_REL = 1000
RELW = 1024
L = 16
NC = 2
NS = 16
NW = NC * NS
OWN = 31232
WBUF = 31744
TAIL0 = (NW - 1) * OWN + WBUF
NTAIL = N_ENT - TAIL0
CW = 3968
NCHUNK = WBUF // CW
BW = 4096
CAP = 1024
CCAP = 256
SLOTS = NCHUNK * CCAP
OUTN = B + NW * SLOTS
CHUNK = 4096


def _score_body(s_idx, r_idx, o_idx, ent_t, rel_flat, tail_flat,
                out_s, out_ro,
                idx_e, idx_r, s_e, s_b, o_e, o_b, o_r,
                s2_e, s2_b, o2_e, o2_b, o2_r, b2, cnts,
                blk_a, blk_b, rel_a, rel_b, tailbuf, acc_s, acc_ro,
                zerobuf, sh_s, sh_ro,
                sem, sem_a, sem_b, sem_ra, sem_rb):
    wid = lax.axis_index("s") * NC + lax.axis_index("c")
    lo = wid * OWN
    hi = jnp.where(wid == NW - 1, N_ENT, lo + OWN)
    base = lo
    iota = lax.iota(jnp.int32, L)
    zero_i = jnp.zeros((L,), jnp.int32)
    zero_f = jnp.zeros((L,), jnp.float32)

    cp_tail = pltpu.async_copy(tail_flat, tailbuf, sem)

    def init1(v, carry):
        s_e[pl.ds(v * L, L)] = zero_i
        o_e[pl.ds(v * L, L)] = zero_i
        o_r[pl.ds(v * L, L)] = zero_i
        pad = B + ((v * L + iota) & 63)
        s_b[pl.ds(v * L, L)] = pad
        o_b[pl.ds(v * L, L)] = pad
        return carry

    lax.fori_loop(0, CAP // L, init1, jnp.int32(0))

    def init2(v, carry):
        s2_e[pl.ds(v * L, L)] = zero_i
        o2_e[pl.ds(v * L, L)] = zero_i
        o2_r[pl.ds(v * L, L)] = zero_i
        acc_s[pl.ds(v * L, L)] = zero_f
        acc_ro[pl.ds(v * L, L)] = zero_f
        pad = B + ((v * L + iota) & 63)
        s2_b[pl.ds(v * L, L)] = pad
        o2_b[pl.ds(v * L, L)] = pad
        return carry

    lax.fori_loop(0, SLOTS // L, init2, jnp.int32(0))

    def initz(v, carry):
        zerobuf[pl.ds(v * L, L)] = zero_f
        return carry

    lax.fori_loop(0, 4096 // L, initz, jnp.int32(0))

    @pl.when(lax.axis_index("s") == 0)
    def _zero_shared():
        for k in range(4):
            pltpu.sync_copy(zerobuf, sh_s.at[pl.ds(k * 4096, 4096)])
            pltpu.sync_copy(zerobuf, sh_ro.at[pl.ds(k * 4096, 4096)])
        pltpu.sync_copy(zerobuf.at[pl.ds(0, 64)], sh_s.at[pl.ds(B, 64)])
        pltpu.sync_copy(zerobuf.at[pl.ds(0, 64)], sh_ro.at[pl.ds(B, 64)])

    plsc.subcore_barrier()

    def schunk(c, cs):
        pltpu.sync_copy(s_idx.at[pl.ds(c * CHUNK, CHUNK)], idx_e)

        def svec(v, cnt):
            e = idx_e[pl.ds(v * L, L)]
            m = (e >= lo) & (e < hi)
            n = plsc.all_reduce_population_count(m)[0]
            plsc.store_compressed(s_e.at[pl.ds(cnt, L)], e - base, mask=m)
            bvec = c * CHUNK + v * L + iota
            plsc.store_compressed(s_b.at[pl.ds(cnt, L)], bvec, mask=m)
            return cnt + n

        return lax.fori_loop(0, CHUNK // L, svec, cs)

    cnt_s = lax.fori_loop(0, B // CHUNK, schunk, jnp.int32(0))

    def ochunk(c, co):
        pltpu.sync_copy(o_idx.at[pl.ds(c * CHUNK, CHUNK)], idx_e)
        pltpu.sync_copy(r_idx.at[pl.ds(c * CHUNK, CHUNK)], idx_r)

        def ovec(v, cnt):
            e = idx_e[pl.ds(v * L, L)]
            m = (e >= lo) & (e < hi)
            n = plsc.all_reduce_population_count(m)[0]
            plsc.store_compressed(o_e.at[pl.ds(cnt, L)], e - base, mask=m)
            bvec = c * CHUNK + v * L + iota
            plsc.store_compressed(o_b.at[pl.ds(cnt, L)], bvec, mask=m)
            rho = idx_r[pl.ds(v * L, L)]
            plsc.store_compressed(o_r.at[pl.ds(cnt, L)], rho, mask=m)
            return cnt + n

        return lax.fori_loop(0, CHUNK // L, ovec, co)

    cnt_o = lax.fori_loop(0, B // CHUNK, ochunk, jnp.int32(0))

    nsv = (cnt_s + L - 1) >> 4
    nov = (cnt_o + L - 1) >> 4

    def rebucket(cc, carry):
        clo = cc * CW
        chi = jnp.where(cc == NCHUNK - 1, jnp.int32(2 ** 30), clo + CW)

        def rvec_s(v, cnt):
            el = s_e[pl.ds(v * L, L)]
            m = (el >= clo) & (el < chi)
            n = plsc.all_reduce_population_count(m)[0]
            plsc.store_compressed(
                s2_e.at[pl.ds(cc * CCAP + cnt, L)], el - clo, mask=m)
            bv = s_b[pl.ds(v * L, L)]
            plsc.store_compressed(
                s2_b.at[pl.ds(cc * CCAP + cnt, L)], bv, mask=m)
            return cnt + n

        cnts[cc] = lax.fori_loop(0, nsv, rvec_s, jnp.int32(0))

        def rvec_o(v, cnt):
            el = o_e[pl.ds(v * L, L)]
            m = (el >= clo) & (el < chi)
            n = plsc.all_reduce_population_count(m)[0]
            plsc.store_compressed(
                o2_e.at[pl.ds(cc * CCAP + cnt, L)], el - clo, mask=m)
            bv = o_b[pl.ds(v * L, L)]
            plsc.store_compressed(
                o2_b.at[pl.ds(cc * CCAP + cnt, L)], bv, mask=m)
            rv = o_r[pl.ds(v * L, L)]
            plsc.store_compressed(
                o2_r.at[pl.ds(cc * CCAP + cnt, L)], rv, mask=m)
            return cnt + n

        cnts[NCHUNK + cc] = lax.fori_loop(0, nov, rvec_o, jnp.int32(0))
        return carry

    lax.fori_loop(0, NCHUNK, rebucket, jnp.int32(0))
    cp_tail.wait()

    blks = (blk_a, blk_b)
    rels = (rel_a, rel_b)
    sems = (sem_a, sem_b)

    rsems = (sem_ra, sem_rb)

    def rel_copy(rr, start):
        ctor = pltpu.async_copy if start else pltpu.make_async_copy
        return ctor(
            rel_flat.at[pl.ds(wid * (D * RELW) + rr * (8 * RELW), 8 * RELW)],
            rels[rr % 2], rsems[rr % 2])

    def copies(i, p, start):
        rr = i >> 3
        cc = i & 7
        ctor = pltpu.async_copy if start else pltpu.make_async_copy
        h = ctor(ent_t.at[pl.ds(rr * 8, 8), pl.ds(base + cc * CW, CW)],
                 blks[p].at[pl.ds(0, 8), pl.ds(0, CW)], sems[p])
        return h

    def wait_copies(i, p):
        copies(i, p, False).wait()

    def compute(i, p, rp):
        rr = i >> 3
        cc = i & 7
        blk = blks[p]
        rel = rels[rp]

        def tmove(dd, carry):
            for t in range(NTAIL // L):
                blk[dd, pl.ds(CW + t * L, L)] = (
                    tailbuf[pl.ds((rr * 8 + dd) * 128 + t * L, L)])
            return carry

        lax.fori_loop(0, 8, tmove, jnp.int32(0))

        nscv = (cnts[cc] + L - 1) >> 4
        nocv = (cnts[NCHUNK + cc] + L - 1) >> 4

        def sv(v, carry2):
            el = s2_e[pl.ds(cc * CCAP + v * L, L)]
            t = plsc.load_gather(blk, [iota * 0, el])
            for dd in range(1, 8):
                t = t + plsc.load_gather(blk, [iota * 0 + dd, el])
            plsc.addupdate(acc_s.at[pl.ds(cc * CCAP + v * L, L)], t)
            return carry2

        lax.fori_loop(0, nscv, sv, jnp.int32(0))

        def ov(v, carry2):
            el = o2_e[pl.ds(cc * CCAP + v * L, L)]
            rho = o2_r[pl.ds(cc * CCAP + v * L, L)]
            t = (plsc.load_gather(blk, [iota * 0, el])
                 * plsc.load_gather(rel, [rho]))
            for dd in range(1, 8):
                t = t + (plsc.load_gather(blk, [iota * 0 + dd, el])
                         * plsc.load_gather(rel, [rho + dd * RELW]))
            plsc.addupdate(acc_ro.at[pl.ds(cc * CCAP + v * L, L)], t)
            return carry2

        lax.fori_loop(0, nocv, ov, jnp.int32(0))

    copies(jnp.int32(0), 0, True)
    rel_copy(0, True)
    for rr in range(8):
        if rr + 1 < 8:
            rel_copy(rr + 1, True)
        rel_copy(rr, False).wait()

        def pair(j, carry, rr=rr):
            i0 = rr * 8 + 2 * j
            i1 = i0 + 1
            copies(i1, 1, True)
            wait_copies(i0, 0)
            compute(i0, 0, rr & 1)
            copies(jnp.minimum(i0 + 2, D - 2), 0, True)
            wait_copies(i1, 1)
            compute(i1, 1, rr & 1)
            return carry

        lax.fori_loop(0, 4, pair, jnp.int32(0))
    wait_copies(jnp.int32(D - 2), 0)

    NB = SLOTS // 128
    for k in range(NB):
        for t in range(128 // L):
            b2[k, pl.ds(t * L, L)] = s2_b[pl.ds(k * 128 + t * L, L)]
            b2[k + NB, pl.ds(t * L, L)] = o2_b[pl.ds(k * 128 + t * L, L)]
    for k in range(NB):
        pltpu.sync_copy(acc_s.at[pl.ds(k * 128, 128)],
                        sh_s.at[b2.at[k]], add=True)
        pltpu.sync_copy(acc_ro.at[pl.ds(k * 128, 128)],
                        sh_ro.at[b2.at[k + NB]], add=True)
    plsc.subcore_barrier()
    sid = lax.axis_index("s")
    cid = lax.axis_index("c")

    @pl.when(sid < 8)
    def _writeback():
        off = sid * (B // 8)
        pltpu.sync_copy(sh_s.at[pl.ds(off, B // 8)],
                        out_s.at[cid, pl.ds(off, B // 8)])
        pltpu.sync_copy(sh_ro.at[pl.ds(off, B // 8)],
                        out_ro.at[cid, pl.ds(off, B // 8)])


def _mul_body(a0_ref, a1_ref, b0_ref, b1_ref, o_ref):
    o_ref[...] = ((a0_ref[...] + a1_ref[...])
                  * (b0_ref[...] + b1_ref[...]))


@jax.jit
def kernel(subjects, relations, objects, entity_table, relation_table):
    s = subjects.reshape(-1).astype(jnp.int32)
    r = relations.reshape(-1).astype(jnp.int32)
    o = objects.reshape(-1).astype(jnp.int32)
    ent_t = entity_table.T
    rel_flat = jnp.tile(jnp.pad(relation_table.T,
                                ((0, 0), (0, RELW - N_REL))).reshape(-1), NW)
    tail_flat = jnp.pad(entity_table[TAIL0:].T,
                        ((0, 0), (0, 128 - NTAIL))).reshape(-1)
    mesh = plsc.VectorSubcoreMesh(core_axis_name="c", subcore_axis_name="s")
    run = functools.partial(
        pl.kernel,
        mesh=mesh,
        compiler_params=pltpu.CompilerParams(needs_layout_passes=False),
        out_type=(jax.ShapeDtypeStruct((NC, B), jnp.float32),
                  jax.ShapeDtypeStruct((NC, B), jnp.float32)),
        scratch_types=[
            pltpu.VMEM((CHUNK,), jnp.int32),
            pltpu.VMEM((CHUNK,), jnp.int32),
            pltpu.VMEM((CAP,), jnp.int32),
            pltpu.VMEM((CAP,), jnp.int32),
            pltpu.VMEM((CAP,), jnp.int32),
            pltpu.VMEM((CAP,), jnp.int32),
            pltpu.VMEM((CAP,), jnp.int32),
            pltpu.VMEM((SLOTS,), jnp.int32),
            pltpu.VMEM((SLOTS,), jnp.int32),
            pltpu.VMEM((SLOTS,), jnp.int32),
            pltpu.VMEM((SLOTS,), jnp.int32),
            pltpu.VMEM((SLOTS,), jnp.int32),
            pltpu.VMEM((2 * SLOTS // 128, 128), jnp.int32),
            pltpu.SMEM((2 * NCHUNK,), jnp.int32),
            pltpu.VMEM((8, BW), jnp.float32),
            pltpu.VMEM((8, BW), jnp.float32),
            pltpu.VMEM((8 * RELW,), jnp.float32),
            pltpu.VMEM((8 * RELW,), jnp.float32),
            pltpu.VMEM((D * 128,), jnp.float32),
            pltpu.VMEM((SLOTS,), jnp.float32),
            pltpu.VMEM((SLOTS,), jnp.float32),
            pltpu.VMEM((4096,), jnp.float32),
            pltpu.VMEM_SHARED((B + 64,), jnp.float32),
            pltpu.VMEM_SHARED((B + 64,), jnp.float32),
            pltpu.SemaphoreType.DMA,
            pltpu.SemaphoreType.DMA,
            pltpu.SemaphoreType.DMA,
            pltpu.SemaphoreType.DMA,
            pltpu.SemaphoreType.DMA,
        ],
    )(_score_body)
    out_s, out_ro = run(s, r, o, ent_t, rel_flat, tail_flat)
    scores = pl.pallas_call(
        _mul_body,
        out_shape=jax.ShapeDtypeStruct((128, 128), jnp.float32),
    )(out_s[0].reshape(128, 128), out_s[1].reshape(128, 128),
      out_ro[0].reshape(128, 128), out_ro[1].reshape(128, 128))
    return scores.reshape(B, 1)


# speedup vs baseline: 3.0652x; 1.0230x over previous
import functools

import jax
import jax.numpy as jnp
from jax import lax
from jax.experimental import pallas as pl
from jax.experimental.pallas import tpu as pltpu
from jax.experimental.pallas import tpu_sc as plsc

B = 16384
D = 64
N_ENT = 1000000
N_REL = 1000
RELW = 1024
L = 16
NC = 2
NS = 16
NW = NC * NS
OWN = 31232
WBUF = 31744
TAIL0 = (NW - 1) * OWN + WBUF
NTAIL = N_ENT - TAIL0
CW = 3968
NCHUNK = WBUF // CW
BW = 4096
CAP = 1024
CCAP = 256
SLOTS = NCHUNK * CCAP
OUTN = B + NW * SLOTS
CHUNK = 2048


def _score_body(s_idx, r_idx, o_idx, ent_t, rel_flat, tail_flat,
                out_s, out_ro,
                idx_s, idx_o, idx_r, s_e, s_b, o_e, o_b, o_r,
                s2_e, s2_b, o2_e, o2_b, o2_r, b2, cnts,
                blk_a, blk_b, rel_a, rel_b, tailbuf, acc_s, acc_ro,
                zerobuf, sh_s, sh_ro,
                sem, sem_a, sem_b, sem_ra, sem_rb):
    wid = lax.axis_index("s") * NC + lax.axis_index("c")
    lo = wid * OWN
    hi = jnp.where(wid == NW - 1, N_ENT, lo + OWN)
    base = lo
    iota = lax.iota(jnp.int32, L)
    zero_i = jnp.zeros((L,), jnp.int32)
    zero_f = jnp.zeros((L,), jnp.float32)

    cp_tail = pltpu.async_copy(tail_flat, tailbuf, sem)

    def init1(v, carry):
        s_e[pl.ds(v * L, L)] = zero_i
        o_e[pl.ds(v * L, L)] = zero_i
        o_r[pl.ds(v * L, L)] = zero_i
        pad = B + ((v * L + iota) & 63)
        s_b[pl.ds(v * L, L)] = pad
        o_b[pl.ds(v * L, L)] = pad
        return carry

    lax.fori_loop(0, CAP // L, init1, jnp.int32(0))

    def init2(v, carry):
        s2_e[pl.ds(v * L, L)] = zero_i
        o2_e[pl.ds(v * L, L)] = zero_i
        o2_r[pl.ds(v * L, L)] = zero_i
        acc_s[pl.ds(v * L, L)] = zero_f
        acc_ro[pl.ds(v * L, L)] = zero_f
        pad = B + ((v * L + iota) & 63)
        s2_b[pl.ds(v * L, L)] = pad
        o2_b[pl.ds(v * L, L)] = pad
        return carry

    lax.fori_loop(0, SLOTS // L, init2, jnp.int32(0))

    def initz(v, carry):
        zerobuf[pl.ds(v * L, L)] = zero_f
        return carry

    lax.fori_loop(0, 4096 // L, initz, jnp.int32(0))

    @pl.when(lax.axis_index("s") == 0)
    def _zero_shared():
        for k in range(4):
            pltpu.sync_copy(zerobuf, sh_s.at[pl.ds(k * 4096, 4096)])
            pltpu.sync_copy(zerobuf, sh_ro.at[pl.ds(k * 4096, 4096)])
        pltpu.sync_copy(zerobuf.at[pl.ds(0, 64)], sh_s.at[pl.ds(B, 64)])
        pltpu.sync_copy(zerobuf.at[pl.ds(0, 64)], sh_ro.at[pl.ds(B, 64)])

    plsc.subcore_barrier()

    def p1chunk(c, cnts_io):
        pltpu.sync_copy(s_idx.at[pl.ds(c * CHUNK, CHUNK)], idx_s)
        pltpu.sync_copy(o_idx.at[pl.ds(c * CHUNK, CHUNK)], idx_o)
        pltpu.sync_copy(r_idx.at[pl.ds(c * CHUNK, CHUNK)], idx_r)

        def vec(v, cnts_io2):
            cs, co = cnts_io2
            bvec = c * CHUNK + v * L + iota
            e_s = idx_s[pl.ds(v * L, L)]
            e_o = idx_o[pl.ds(v * L, L)]
            m_s = (e_s >= lo) & (e_s < hi)
            m_o = (e_o >= lo) & (e_o < hi)
            n_s = plsc.all_reduce_population_count(m_s)[0]
            n_o = plsc.all_reduce_population_count(m_o)[0]
            plsc.store_compressed(s_e.at[pl.ds(cs, L)], e_s - base, mask=m_s)
            plsc.store_compressed(s_b.at[pl.ds(cs, L)], bvec, mask=m_s)
            plsc.store_compressed(o_e.at[pl.ds(co, L)], e_o - base, mask=m_o)
            plsc.store_compressed(o_b.at[pl.ds(co, L)], bvec, mask=m_o)
            rho = idx_r[pl.ds(v * L, L)]
            plsc.store_compressed(o_r.at[pl.ds(co, L)], rho, mask=m_o)
            return (cs + n_s, co + n_o)

        return lax.fori_loop(0, CHUNK // L, vec, cnts_io)

    cnt_s, cnt_o = lax.fori_loop(0, B // CHUNK, p1chunk,
                                 (jnp.int32(0), jnp.int32(0)))

    nsv = (cnt_s + L - 1) >> 4
    nov = (cnt_o + L - 1) >> 4

    def rebucket(cc, carry):
        clo = cc * CW
        chi = jnp.where(cc == NCHUNK - 1, jnp.int32(2 ** 30), clo + CW)

        def rvec_s(v, cnt):
            el = s_e[pl.ds(v * L, L)]
            m = (el >= clo) & (el < chi)
            n = plsc.all_reduce_population_count(m)[0]
            plsc.store_compressed(
                s2_e.at[pl.ds(cc * CCAP + cnt, L)], el - clo, mask=m)
            bv = s_b[pl.ds(v * L, L)]
            plsc.store_compressed(
                s2_b.at[pl.ds(cc * CCAP + cnt, L)], bv, mask=m)
            return cnt + n

        cnts[cc] = lax.fori_loop(0, nsv, rvec_s, jnp.int32(0))

        def rvec_o(v, cnt):
            el = o_e[pl.ds(v * L, L)]
            m = (el >= clo) & (el < chi)
            n = plsc.all_reduce_population_count(m)[0]
            plsc.store_compressed(
                o2_e.at[pl.ds(cc * CCAP + cnt, L)], el - clo, mask=m)
            bv = o_b[pl.ds(v * L, L)]
            plsc.store_compressed(
                o2_b.at[pl.ds(cc * CCAP + cnt, L)], bv, mask=m)
            rv = o_r[pl.ds(v * L, L)]
            plsc.store_compressed(
                o2_r.at[pl.ds(cc * CCAP + cnt, L)], rv, mask=m)
            return cnt + n

        cnts[NCHUNK + cc] = lax.fori_loop(0, nov, rvec_o, jnp.int32(0))
        return carry

    lax.fori_loop(0, NCHUNK, rebucket, jnp.int32(0))
    cp_tail.wait()

    blks = (blk_a, blk_b)
    rels = (rel_a, rel_b)
    sems = (sem_a, sem_b)

    rsems = (sem_ra, sem_rb)

    def rel_copy(rr, start):
        ctor = pltpu.async_copy if start else pltpu.make_async_copy
        return ctor(
            rel_flat.at[pl.ds(wid * (D * RELW) + rr * (8 * RELW), 8 * RELW)],
            rels[rr % 2], rsems[rr % 2])

    def copies(i, p, start):
        rr = i >> 3
        cc = i & 7
        ctor = pltpu.async_copy if start else pltpu.make_async_copy
        h = ctor(ent_t.at[pl.ds(rr * 8, 8), pl.ds(base + cc * CW, CW)],
                 blks[p].at[pl.ds(0, 8), pl.ds(0, CW)], sems[p])
        return h

    def wait_copies(i, p):
        copies(i, p, False).wait()

    def compute(i, p, rp):
        rr = i >> 3
        cc = i & 7
        blk = blks[p]
        rel = rels[rp]

        def tmove(dd, carry):
            for t in range(NTAIL // L):
                blk[dd, pl.ds(CW + t * L, L)] = (
                    tailbuf[pl.ds((rr * 8 + dd) * 128 + t * L, L)])
            return carry

        lax.fori_loop(0, 8, tmove, jnp.int32(0))

        nscv = (cnts[cc] + L - 1) >> 4
        nocv = (cnts[NCHUNK + cc] + L - 1) >> 4

        def sv(v, carry2):
            el = s2_e[pl.ds(cc * CCAP + v * L, L)]
            t = plsc.load_gather(blk, [iota * 0, el])
            for dd in range(1, 8):
                t = t + plsc.load_gather(blk, [iota * 0 + dd, el])
            plsc.addupdate(acc_s.at[pl.ds(cc * CCAP + v * L, L)], t)
            return carry2

        lax.fori_loop(0, nscv, sv, jnp.int32(0))

        def ov(v, carry2):
            el = o2_e[pl.ds(cc * CCAP + v * L, L)]
            rho = o2_r[pl.ds(cc * CCAP + v * L, L)]
            t = (plsc.load_gather(blk, [iota * 0, el])
                 * plsc.load_gather(rel, [rho]))
            for dd in range(1, 8):
                t = t + (plsc.load_gather(blk, [iota * 0 + dd, el])
                         * plsc.load_gather(rel, [rho + dd * RELW]))
            plsc.addupdate(acc_ro.at[pl.ds(cc * CCAP + v * L, L)], t)
            return carry2

        lax.fori_loop(0, nocv, ov, jnp.int32(0))

    copies(jnp.int32(0), 0, True)
    rel_copy(0, True)
    for rr in range(8):
        if rr + 1 < 8:
            rel_copy(rr + 1, True)
        rel_copy(rr, False).wait()

        def pair(j, carry, rr=rr):
            i0 = rr * 8 + 2 * j
            i1 = i0 + 1
            copies(i1, 1, True)
            wait_copies(i0, 0)
            compute(i0, 0, rr & 1)
            copies(jnp.minimum(i0 + 2, D - 2), 0, True)
            wait_copies(i1, 1)
            compute(i1, 1, rr & 1)
            return carry

        lax.fori_loop(0, 4, pair, jnp.int32(0))
    wait_copies(jnp.int32(D - 2), 0)

    NB = SLOTS // 128
    for k in range(NB):
        for t in range(128 // L):
            b2[k, pl.ds(t * L, L)] = s2_b[pl.ds(k * 128 + t * L, L)]
            b2[k + NB, pl.ds(t * L, L)] = o2_b[pl.ds(k * 128 + t * L, L)]
    for k in range(NB):
        pltpu.sync_copy(acc_s.at[pl.ds(k * 128, 128)],
                        sh_s.at[b2.at[k]], add=True)
        pltpu.sync_copy(acc_ro.at[pl.ds(k * 128, 128)],
                        sh_ro.at[b2.at[k + NB]], add=True)
    plsc.subcore_barrier()
    sid = lax.axis_index("s")
    cid = lax.axis_index("c")

    @pl.when(sid < 8)
    def _writeback():
        off = sid * (B // 8)
        pltpu.sync_copy(sh_s.at[pl.ds(off, B // 8)],
                        out_s.at[cid, pl.ds(off, B // 8)])
        pltpu.sync_copy(sh_ro.at[pl.ds(off, B // 8)],
                        out_ro.at[cid, pl.ds(off, B // 8)])


def _mul_body(a0_ref, a1_ref, b0_ref, b1_ref, o_ref):
    o_ref[...] = ((a0_ref[...] + a1_ref[...])
                  * (b0_ref[...] + b1_ref[...]))


@jax.jit
def kernel(subjects, relations, objects, entity_table, relation_table):
    s = subjects.reshape(-1).astype(jnp.int32)
    r = relations.reshape(-1).astype(jnp.int32)
    o = objects.reshape(-1).astype(jnp.int32)
    ent_t = entity_table.T
    rel_flat = jnp.tile(jnp.pad(relation_table.T,
                                ((0, 0), (0, RELW - N_REL))).reshape(-1), NW)
    tail_flat = jnp.pad(entity_table[TAIL0:].T,
                        ((0, 0), (0, 128 - NTAIL))).reshape(-1)
    mesh = plsc.VectorSubcoreMesh(core_axis_name="c", subcore_axis_name="s")
    run = functools.partial(
        pl.kernel,
        mesh=mesh,
        compiler_params=pltpu.CompilerParams(needs_layout_passes=False),
        out_type=(jax.ShapeDtypeStruct((NC, B), jnp.float32),
                  jax.ShapeDtypeStruct((NC, B), jnp.float32)),
        scratch_types=[
            pltpu.VMEM((CHUNK,), jnp.int32),
            pltpu.VMEM((CHUNK,), jnp.int32),
            pltpu.VMEM((CHUNK,), jnp.int32),
            pltpu.VMEM((CAP,), jnp.int32),
            pltpu.VMEM((CAP,), jnp.int32),
            pltpu.VMEM((CAP,), jnp.int32),
            pltpu.VMEM((CAP,), jnp.int32),
            pltpu.VMEM((CAP,), jnp.int32),
            pltpu.VMEM((SLOTS,), jnp.int32),
            pltpu.VMEM((SLOTS,), jnp.int32),
            pltpu.VMEM((SLOTS,), jnp.int32),
            pltpu.VMEM((SLOTS,), jnp.int32),
            pltpu.VMEM((SLOTS,), jnp.int32),
            pltpu.VMEM((2 * SLOTS // 128, 128), jnp.int32),
            pltpu.SMEM((2 * NCHUNK,), jnp.int32),
            pltpu.VMEM((8, BW), jnp.float32),
            pltpu.VMEM((8, BW), jnp.float32),
            pltpu.VMEM((8 * RELW,), jnp.float32),
            pltpu.VMEM((8 * RELW,), jnp.float32),
            pltpu.VMEM((D * 128,), jnp.float32),
            pltpu.VMEM((SLOTS,), jnp.float32),
            pltpu.VMEM((SLOTS,), jnp.float32),
            pltpu.VMEM((4096,), jnp.float32),
            pltpu.VMEM_SHARED((B + 64,), jnp.float32),
            pltpu.VMEM_SHARED((B + 64,), jnp.float32),
            pltpu.SemaphoreType.DMA,
            pltpu.SemaphoreType.DMA,
            pltpu.SemaphoreType.DMA,
            pltpu.SemaphoreType.DMA,
            pltpu.SemaphoreType.DMA,
        ],
    )(_score_body)
    out_s, out_ro = run(s, r, o, ent_t, rel_flat, tail_flat)
    scores = pl.pallas_call(
        _mul_body,
        out_shape=jax.ShapeDtypeStruct((128, 128), jnp.float32),
    )(out_s[0].reshape(128, 128), out_s[1].reshape(128, 128),
      out_ro[0].reshape(128, 128), out_ro[1].reshape(128, 128))
    return scores.reshape(B, 1)


# speedup vs baseline: 3.0658x; 1.0002x over previous
import functools

import jax
import jax.numpy as jnp
from jax import lax
from jax.experimental import pallas as pl
from jax.experimental.pallas import tpu as pltpu
from jax.experimental.pallas import tpu_sc as plsc

B = 16384
D = 64
N_ENT = 1000000
N_REL = 1000
RELW = 1024
L = 16
NC = 2
NS = 16
NW = NC * NS
OWN = 31232
WBUF = 31744
TAIL0 = (NW - 1) * OWN + WBUF
NTAIL = N_ENT - TAIL0
CW = 3968
NCHUNK = WBUF // CW
BW = 4096
CAP = 1024
CCAP = 256
SLOTS = NCHUNK * CCAP
CHUNK = 2048


def _score_body(s_idx, r_idx, o_idx, ent_t, rel_flat, tail_flat,
                out_s, out_ro,
                idx_s, idx_o, idx_r, s_e, s_b, o_e, o_b, o_r,
                s2_e, s2_b, o2_e, o2_b, o2_r, b2, cnts,
                blk_a, blk_b, rel_a, rel_b, tailbuf, acc_s, acc_ro,
                zerobuf, sh_s, sh_ro,
                sem, sem_a, sem_b, sem_ra, sem_rb):
    wid = lax.axis_index("s") * NC + lax.axis_index("c")
    lo = wid * OWN
    hi = jnp.where(wid == NW - 1, N_ENT, lo + OWN)
    base = lo
    iota = lax.iota(jnp.int32, L)
    zero_i = jnp.zeros((L,), jnp.int32)
    zero_f = jnp.zeros((L,), jnp.float32)

    cp_tail = pltpu.async_copy(tail_flat, tailbuf, sem)

    def init1(v, carry):
        s_e[pl.ds(v * L, L)] = zero_i
        o_e[pl.ds(v * L, L)] = zero_i
        o_r[pl.ds(v * L, L)] = zero_i
        pad = B + ((v * L + iota) & 63)
        s_b[pl.ds(v * L, L)] = pad
        o_b[pl.ds(v * L, L)] = pad
        return carry

    lax.fori_loop(0, CAP // L, init1, jnp.int32(0))

    def init2(v, carry):
        s2_e[pl.ds(v * L, L)] = zero_i
        o2_e[pl.ds(v * L, L)] = zero_i
        o2_r[pl.ds(v * L, L)] = zero_i
        acc_s[pl.ds(v * L, L)] = zero_f
        acc_ro[pl.ds(v * L, L)] = zero_f
        pad = B + ((v * L + iota) & 63)
        s2_b[pl.ds(v * L, L)] = pad
        o2_b[pl.ds(v * L, L)] = pad
        return carry

    lax.fori_loop(0, SLOTS // L, init2, jnp.int32(0))

    def initz(v, carry):
        zerobuf[pl.ds(v * L, L)] = zero_f
        return carry

    lax.fori_loop(0, 4096 // L, initz, jnp.int32(0))

    @pl.when(lax.axis_index("s") == 0)
    def _zero_shared():
        for k in range(4):
            pltpu.sync_copy(zerobuf, sh_s.at[pl.ds(k * 4096, 4096)])
            pltpu.sync_copy(zerobuf, sh_ro.at[pl.ds(k * 4096, 4096)])
        pltpu.sync_copy(zerobuf.at[pl.ds(0, 64)], sh_s.at[pl.ds(B, 64)])
        pltpu.sync_copy(zerobuf.at[pl.ds(0, 64)], sh_ro.at[pl.ds(B, 64)])

    plsc.subcore_barrier()

    def p1chunk(c, cnts_io):
        pltpu.sync_copy(s_idx.at[pl.ds(c * CHUNK, CHUNK)], idx_s)
        pltpu.sync_copy(o_idx.at[pl.ds(c * CHUNK, CHUNK)], idx_o)
        pltpu.sync_copy(r_idx.at[pl.ds(c * CHUNK, CHUNK)], idx_r)

        def vec(v, cnts_io2):
            cs, co = cnts_io2
            bvec = c * CHUNK + v * L + iota
            e_s = idx_s[pl.ds(v * L, L)]
            e_o = idx_o[pl.ds(v * L, L)]
            m_s = (e_s >= lo) & (e_s < hi)
            m_o = (e_o >= lo) & (e_o < hi)
            n_s = plsc.all_reduce_population_count(m_s)[0]
            n_o = plsc.all_reduce_population_count(m_o)[0]
            plsc.store_compressed(s_e.at[pl.ds(cs, L)], e_s - base, mask=m_s)
            plsc.store_compressed(s_b.at[pl.ds(cs, L)], bvec, mask=m_s)
            plsc.store_compressed(o_e.at[pl.ds(co, L)], e_o - base, mask=m_o)
            plsc.store_compressed(o_b.at[pl.ds(co, L)], bvec, mask=m_o)
            rho = idx_r[pl.ds(v * L, L)]
            plsc.store_compressed(o_r.at[pl.ds(co, L)], rho, mask=m_o)
            return (cs + n_s, co + n_o)

        return lax.fori_loop(0, CHUNK // L, vec, cnts_io)

    cnt_s, cnt_o = lax.fori_loop(0, B // CHUNK, p1chunk,
                                 (jnp.int32(0), jnp.int32(0)))

    nsv = (cnt_s + L - 1) >> 4
    nov = (cnt_o + L - 1) >> 4

    def rebucket(cc, carry):
        clo = cc * CW
        chi = jnp.where(cc == NCHUNK - 1, jnp.int32(2 ** 30), clo + CW)

        def rvec_s(v, cnt):
            el = s_e[pl.ds(v * L, L)]
            m = (el >= clo) & (el < chi)
            n = plsc.all_reduce_population_count(m)[0]
            plsc.store_compressed(
                s2_e.at[pl.ds(cc * CCAP + cnt, L)], el - clo, mask=m)
            bv = s_b[pl.ds(v * L, L)]
            plsc.store_compressed(
                s2_b.at[pl.ds(cc * CCAP + cnt, L)], bv, mask=m)
            return cnt + n

        cnts[cc] = lax.fori_loop(0, nsv, rvec_s, jnp.int32(0))

        def rvec_o(v, cnt):
            el = o_e[pl.ds(v * L, L)]
            m = (el >= clo) & (el < chi)
            n = plsc.all_reduce_population_count(m)[0]
            plsc.store_compressed(
                o2_e.at[pl.ds(cc * CCAP + cnt, L)], el - clo, mask=m)
            bv = o_b[pl.ds(v * L, L)]
            plsc.store_compressed(
                o2_b.at[pl.ds(cc * CCAP + cnt, L)], bv, mask=m)
            rv = o_r[pl.ds(v * L, L)]
            plsc.store_compressed(
                o2_r.at[pl.ds(cc * CCAP + cnt, L)], rv, mask=m)
            return cnt + n

        cnts[NCHUNK + cc] = lax.fori_loop(0, nov, rvec_o, jnp.int32(0))
        return carry

    lax.fori_loop(0, NCHUNK, rebucket, jnp.int32(0))
    cp_tail.wait()

    blks = (blk_a, blk_b)
    rels = (rel_a, rel_b)
    sems = (sem_a, sem_b)

    rsems = (sem_ra, sem_rb)

    def rel_copy(rr, start):
        ctor = pltpu.async_copy if start else pltpu.make_async_copy
        return ctor(
            rel_flat.at[pl.ds(wid * (D * RELW) + rr * (8 * RELW), 8 * RELW)],
            rels[rr % 2], rsems[rr % 2])

    def copies(i, p, start):
        rr = i >> 3
        cc = i & 7
        ctor = pltpu.async_copy if start else pltpu.make_async_copy
        h = ctor(ent_t.at[pl.ds(rr * 8, 8), pl.ds(base + cc * CW, CW)],
                 blks[p].at[pl.ds(0, 8), pl.ds(0, CW)], sems[p])
        return h

    def wait_copies(i, p):
        copies(i, p, False).wait()

    def compute(i, p, rp):
        rr = i >> 3
        cc = i & 7
        blk = blks[p]
        rel = rels[rp]

        def tmove(dd, carry):
            for t in range(NTAIL // L):
                blk[dd, pl.ds(CW + t * L, L)] = (
                    tailbuf[pl.ds((rr * 8 + dd) * 128 + t * L, L)])
            return carry

        lax.fori_loop(0, 8, tmove, jnp.int32(0))

        nscv = (cnts[cc] + L - 1) >> 4
        nocv = (cnts[NCHUNK + cc] + L - 1) >> 4

        def sv(v, carry2):
            el = s2_e[pl.ds(cc * CCAP + v * L, L)]
            t = plsc.load_gather(blk, [iota * 0, el])
            for dd in range(1, 8):
                t = t + plsc.load_gather(blk, [iota * 0 + dd, el])
            plsc.addupdate(acc_s.at[pl.ds(cc * CCAP + v * L, L)], t)
            return carry2

        lax.fori_loop(0, nscv, sv, jnp.int32(0))

        def ov(v, carry2):
            el = o2_e[pl.ds(cc * CCAP + v * L, L)]
            rho = o2_r[pl.ds(cc * CCAP + v * L, L)]
            t = (plsc.load_gather(blk, [iota * 0, el])
                 * plsc.load_gather(rel, [rho]))
            for dd in range(1, 8):
                t = t + (plsc.load_gather(blk, [iota * 0 + dd, el])
                         * plsc.load_gather(rel, [rho + dd * RELW]))
            plsc.addupdate(acc_ro.at[pl.ds(cc * CCAP + v * L, L)], t)
            return carry2

        lax.fori_loop(0, nocv, ov, jnp.int32(0))

    copies(jnp.int32(0), 0, True)
    rel_copy(0, True)
    for rr in range(8):
        if rr + 1 < 8:
            rel_copy(rr + 1, True)
        rel_copy(rr, False).wait()

        def pair(j, carry, rr=rr):
            i0 = rr * 8 + 2 * j
            i1 = i0 + 1
            copies(i1, 1, True)
            wait_copies(i0, 0)
            compute(i0, 0, rr & 1)
            copies(jnp.minimum(i0 + 2, D - 2), 0, True)
            wait_copies(i1, 1)
            compute(i1, 1, rr & 1)
            return carry

        lax.fori_loop(0, 4, pair, jnp.int32(0))
    wait_copies(jnp.int32(D - 2), 0)

    NB = SLOTS // 128
    for k in range(NB):
        for t in range(128 // L):
            b2[k, pl.ds(t * L, L)] = s2_b[pl.ds(k * 128 + t * L, L)]
            b2[k + NB, pl.ds(t * L, L)] = o2_b[pl.ds(k * 128 + t * L, L)]
    for k in range(NB):
        pltpu.sync_copy(acc_s.at[pl.ds(k * 128, 128)],
                        sh_s.at[b2.at[k]], add=True)
        pltpu.sync_copy(acc_ro.at[pl.ds(k * 128, 128)],
                        sh_ro.at[b2.at[k + NB]], add=True)
    plsc.subcore_barrier()
    sid = lax.axis_index("s")
    cid = lax.axis_index("c")

    @pl.when(sid < 8)
    def _writeback():
        off = sid * (B // 8)
        pltpu.sync_copy(sh_s.at[pl.ds(off, B // 8)],
                        out_s.at[cid, pl.ds(off, B // 8)])
        pltpu.sync_copy(sh_ro.at[pl.ds(off, B // 8)],
                        out_ro.at[cid, pl.ds(off, B // 8)])


def _mul_body(a0_ref, a1_ref, b0_ref, b1_ref, o_ref):
    o_ref[...] = ((a0_ref[...] + a1_ref[...])
                  * (b0_ref[...] + b1_ref[...]))


@jax.jit
def kernel(subjects, relations, objects, entity_table, relation_table):
    s = subjects.reshape(-1).astype(jnp.int32)
    r = relations.reshape(-1).astype(jnp.int32)
    o = objects.reshape(-1).astype(jnp.int32)
    ent_t = entity_table.T
    rel_flat = jnp.tile(jnp.pad(relation_table.T,
                                ((0, 0), (0, RELW - N_REL))).reshape(-1), NW)
    tail_flat = jnp.pad(entity_table[TAIL0:].T,
                        ((0, 0), (0, 128 - NTAIL))).reshape(-1)
    mesh = plsc.VectorSubcoreMesh(core_axis_name="c", subcore_axis_name="s")
    run = functools.partial(
        pl.kernel,
        mesh=mesh,
        compiler_params=pltpu.CompilerParams(needs_layout_passes=False),
        out_type=(jax.ShapeDtypeStruct((NC, B), jnp.float32),
                  jax.ShapeDtypeStruct((NC, B), jnp.float32)),
        scratch_types=[
            pltpu.VMEM((CHUNK,), jnp.int32),
            pltpu.VMEM((CHUNK,), jnp.int32),
            pltpu.VMEM((CHUNK,), jnp.int32),
            pltpu.VMEM((CAP,), jnp.int32),
            pltpu.VMEM((CAP,), jnp.int32),
            pltpu.VMEM((CAP,), jnp.int32),
            pltpu.VMEM((CAP,), jnp.int32),
            pltpu.VMEM((CAP,), jnp.int32),
            pltpu.VMEM((SLOTS,), jnp.int32),
            pltpu.VMEM((SLOTS,), jnp.int32),
            pltpu.VMEM((SLOTS,), jnp.int32),
            pltpu.VMEM((SLOTS,), jnp.int32),
            pltpu.VMEM((SLOTS,), jnp.int32),
            pltpu.VMEM((2 * SLOTS // 128, 128), jnp.int32),
            pltpu.SMEM((2 * NCHUNK,), jnp.int32),
            pltpu.VMEM((8, BW), jnp.float32),
            pltpu.VMEM((8, BW), jnp.float32),
            pltpu.VMEM((8 * RELW,), jnp.float32),
            pltpu.VMEM((8 * RELW,), jnp.float32),
            pltpu.VMEM((D * 128,), jnp.float32),
            pltpu.VMEM((SLOTS,), jnp.float32),
            pltpu.VMEM((SLOTS,), jnp.float32),
            pltpu.VMEM((4096,), jnp.float32),
            pltpu.VMEM_SHARED((B + 64,), jnp.float32),
            pltpu.VMEM_SHARED((B + 64,), jnp.float32),
            pltpu.SemaphoreType.DMA,
            pltpu.SemaphoreType.DMA,
            pltpu.SemaphoreType.DMA,
            pltpu.SemaphoreType.DMA,
            pltpu.SemaphoreType.DMA,
        ],
    )(_score_body)
    out_s, out_ro = run(s, r, o, ent_t, rel_flat, tail_flat)
    scores = pl.pallas_call(
        _mul_body,
        out_shape=jax.ShapeDtypeStruct((128, 128), jnp.float32),
    )(out_s[0].reshape(128, 128), out_s[1].reshape(128, 128),
      out_ro[0].reshape(128, 128), out_ro[1].reshape(128, 128))
    return scores.reshape(B, 1)


# speedup vs baseline: 3.2978x; 1.0757x over previous
import functools

import jax
import jax.numpy as jnp
from jax import lax
from jax.experimental import pallas as pl
from jax.experimental.pallas import tpu as pltpu
from jax.experimental.pallas import tpu_sc as plsc

B = 16384
D = 64
N_ENT = 1000000
N_REL = 1000
RELW = 1024
L = 16
NC = 2
NS = 16
NW = NC * NS
OWN = 31232
WBUF = 31744
TAIL0 = (NW - 1) * OWN + WBUF
NTAIL = N_ENT - TAIL0
CW = 3968
NCHUNK = WBUF // CW
BW = 4096
CAP = 1024
CCAP = 256
SLOTS = NCHUNK * CCAP
CHUNK = 1024


def _score_body(s_idx, r_idx, o_idx, ent_t, rel_flat, tail_flat,
                out_s, out_ro,
                idx_s, idx_o, idx_r, idx_s2, idx_o2, idx_r2,
                s_e, s_b, o_e, o_b, o_r,
                s2_e, s2_b, o2_e, o2_b, o2_r, b2, cnts,
                blk_a, blk_b, rel_a, rel_b, tailbuf, acc_s, acc_ro,
                zerobuf, sh_s, sh_ro,
                sem, sem_a, sem_b, sem_ra, sem_rb, sem_pa, sem_pb):
    wid = lax.axis_index("s") * NC + lax.axis_index("c")
    lo = wid * OWN
    hi = jnp.where(wid == NW - 1, N_ENT, lo + OWN)
    base = lo
    iota = lax.iota(jnp.int32, L)
    zero_i = jnp.zeros((L,), jnp.int32)
    zero_f = jnp.zeros((L,), jnp.float32)

    cp_tail = pltpu.async_copy(tail_flat, tailbuf, sem)

    def init1(v, carry):
        s_e[pl.ds(v * L, L)] = zero_i
        o_e[pl.ds(v * L, L)] = zero_i
        o_r[pl.ds(v * L, L)] = zero_i
        pad = B + ((v * L + iota) & 63)
        s_b[pl.ds(v * L, L)] = pad
        o_b[pl.ds(v * L, L)] = pad
        return carry

    lax.fori_loop(0, CAP // L, init1, jnp.int32(0))

    def init2(v, carry):
        s2_e[pl.ds(v * L, L)] = zero_i
        o2_e[pl.ds(v * L, L)] = zero_i
        o2_r[pl.ds(v * L, L)] = zero_i
        acc_s[pl.ds(v * L, L)] = zero_f
        acc_ro[pl.ds(v * L, L)] = zero_f
        pad = B + ((v * L + iota) & 63)
        s2_b[pl.ds(v * L, L)] = pad
        o2_b[pl.ds(v * L, L)] = pad
        return carry

    lax.fori_loop(0, SLOTS // L, init2, jnp.int32(0))

    def initz(v, carry):
        zerobuf[pl.ds(v * L, L)] = zero_f
        return carry

    lax.fori_loop(0, 4096 // L, initz, jnp.int32(0))

    @pl.when(lax.axis_index("s") == 0)
    def _zero_shared():
        for k in range(4):
            pltpu.sync_copy(zerobuf, sh_s.at[pl.ds(k * 4096, 4096)])
            pltpu.sync_copy(zerobuf, sh_ro.at[pl.ds(k * 4096, 4096)])
        pltpu.sync_copy(zerobuf.at[pl.ds(0, 64)], sh_s.at[pl.ds(B, 64)])
        pltpu.sync_copy(zerobuf.at[pl.ds(0, 64)], sh_ro.at[pl.ds(B, 64)])

    plsc.subcore_barrier()

    NP1 = B // CHUNK
    p1bufs = ((idx_s, idx_o, idx_r), (idx_s2, idx_o2, idx_r2))
    p1sems = (sem_pa, sem_pb)

    def p1stage(c, par, start):
        ctor = pltpu.async_copy if start else pltpu.make_async_copy
        bs, bo, br = p1bufs[par]
        hs = ctor(s_idx.at[pl.ds(c * CHUNK, CHUNK)], bs, p1sems[par])
        ho = ctor(o_idx.at[pl.ds(c * CHUNK, CHUNK)], bo, p1sems[par])
        hr = ctor(r_idx.at[pl.ds(c * CHUNK, CHUNK)], br, p1sems[par])
        return hs, ho, hr

    def p1wait(c, par):
        for h in p1stage(c, par, False):
            h.wait()

    def p1scan(c, par, cnts_io):
        bs, bo, br = p1bufs[par]

        def vec(v, cnts_io2):
            cs, co = cnts_io2
            bvec = c * CHUNK + v * L + iota
            e_s = bs[pl.ds(v * L, L)]
            e_o = bo[pl.ds(v * L, L)]
            m_s = (e_s >= lo) & (e_s < hi)
            m_o = (e_o >= lo) & (e_o < hi)
            n_s = plsc.all_reduce_population_count(m_s)[0]
            n_o = plsc.all_reduce_population_count(m_o)[0]
            plsc.store_compressed(s_e.at[pl.ds(cs, L)], e_s - base, mask=m_s)
            plsc.store_compressed(s_b.at[pl.ds(cs, L)], bvec, mask=m_s)
            plsc.store_compressed(o_e.at[pl.ds(co, L)], e_o - base, mask=m_o)
            plsc.store_compressed(o_b.at[pl.ds(co, L)], bvec, mask=m_o)
            rho = br[pl.ds(v * L, L)]
            plsc.store_compressed(o_r.at[pl.ds(co, L)], rho, mask=m_o)
            return (cs + n_s, co + n_o)

        return lax.fori_loop(0, CHUNK // L, vec, cnts_io)

    p1stage(jnp.int32(0), 0, True)

    def p1pair(j, cnts_io):
        c0 = 2 * j
        c1 = c0 + 1
        p1stage(c1, 1, True)
        p1wait(c0, 0)
        cnts_io = p1scan(c0, 0, cnts_io)
        p1stage(jnp.minimum(c0 + 2, NP1 - 2), 0, True)
        p1wait(c1, 1)
        cnts_io = p1scan(c1, 1, cnts_io)
        return cnts_io

    cnt_s, cnt_o = lax.fori_loop(0, NP1 // 2, p1pair,
                                 (jnp.int32(0), jnp.int32(0)))
    p1wait(jnp.int32(NP1 - 2), 0)

    nsv = (cnt_s + L - 1) >> 4
    nov = (cnt_o + L - 1) >> 4

    def rebucket(cc, carry):
        clo = cc * CW
        chi = jnp.where(cc == NCHUNK - 1, jnp.int32(2 ** 30), clo + CW)

        def rvec_s(v, cnt):
            el = s_e[pl.ds(v * L, L)]
            m = (el >= clo) & (el < chi)
            n = plsc.all_reduce_population_count(m)[0]
            plsc.store_compressed(
                s2_e.at[pl.ds(cc * CCAP + cnt, L)], el - clo, mask=m)
            bv = s_b[pl.ds(v * L, L)]
            plsc.store_compressed(
                s2_b.at[pl.ds(cc * CCAP + cnt, L)], bv, mask=m)
            return cnt + n

        cnts[cc] = lax.fori_loop(0, nsv, rvec_s, jnp.int32(0))

        def rvec_o(v, cnt):
            el = o_e[pl.ds(v * L, L)]
            m = (el >= clo) & (el < chi)
            n = plsc.all_reduce_population_count(m)[0]
            plsc.store_compressed(
                o2_e.at[pl.ds(cc * CCAP + cnt, L)], el - clo, mask=m)
            bv = o_b[pl.ds(v * L, L)]
            plsc.store_compressed(
                o2_b.at[pl.ds(cc * CCAP + cnt, L)], bv, mask=m)
            rv = o_r[pl.ds(v * L, L)]
            plsc.store_compressed(
                o2_r.at[pl.ds(cc * CCAP + cnt, L)], rv, mask=m)
            return cnt + n

        cnts[NCHUNK + cc] = lax.fori_loop(0, nov, rvec_o, jnp.int32(0))
        return carry

    lax.fori_loop(0, NCHUNK, rebucket, jnp.int32(0))
    cp_tail.wait()

    blks = (blk_a, blk_b)
    rels = (rel_a, rel_b)
    sems = (sem_a, sem_b)

    rsems = (sem_ra, sem_rb)

    def rel_copy(rr, start):
        ctor = pltpu.async_copy if start else pltpu.make_async_copy
        return ctor(
            rel_flat.at[pl.ds(wid * (D * RELW) + rr * (8 * RELW), 8 * RELW)],
            rels[rr % 2], rsems[rr % 2])

    def copies(i, p, start):
        rr = i >> 3
        cc = i & 7
        ctor = pltpu.async_copy if start else pltpu.make_async_copy
        h = ctor(ent_t.at[pl.ds(rr * 8, 8), pl.ds(base + cc * CW, CW)],
                 blks[p].at[pl.ds(0, 8), pl.ds(0, CW)], sems[p])
        return h

    def wait_copies(i, p):
        copies(i, p, False).wait()

    def compute(i, p, rp):
        rr = i >> 3
        cc = i & 7
        blk = blks[p]
        rel = rels[rp]

        def tmove(dd, carry):
            for t in range(NTAIL // L):
                blk[dd, pl.ds(CW + t * L, L)] = (
                    tailbuf[pl.ds((rr * 8 + dd) * 128 + t * L, L)])
            return carry

        lax.fori_loop(0, 8, tmove, jnp.int32(0))

        nscv = (cnts[cc] + L - 1) >> 4
        nocv = (cnts[NCHUNK + cc] + L - 1) >> 4

        def sv(v, carry2):
            el = s2_e[pl.ds(cc * CCAP + v * L, L)]
            t = plsc.load_gather(blk, [iota * 0, el])
            for dd in range(1, 8):
                t = t + plsc.load_gather(blk, [iota * 0 + dd, el])
            plsc.addupdate(acc_s.at[pl.ds(cc * CCAP + v * L, L)], t)
            return carry2

        lax.fori_loop(0, nscv, sv, jnp.int32(0))

        def ov(v, carry2):
            el = o2_e[pl.ds(cc * CCAP + v * L, L)]
            rho = o2_r[pl.ds(cc * CCAP + v * L, L)]
            t = (plsc.load_gather(blk, [iota * 0, el])
                 * plsc.load_gather(rel, [rho]))
            for dd in range(1, 8):
                t = t + (plsc.load_gather(blk, [iota * 0 + dd, el])
                         * plsc.load_gather(rel, [rho + dd * RELW]))
            plsc.addupdate(acc_ro.at[pl.ds(cc * CCAP + v * L, L)], t)
            return carry2

        lax.fori_loop(0, nocv, ov, jnp.int32(0))

    copies(jnp.int32(0), 0, True)
    rel_copy(0, True)
    for rr in range(8):
        if rr + 1 < 8:
            rel_copy(rr + 1, True)
        rel_copy(rr, False).wait()

        def pair(j, carry, rr=rr):
            i0 = rr * 8 + 2 * j
            i1 = i0 + 1
            copies(i1, 1, True)
            wait_copies(i0, 0)
            compute(i0, 0, rr & 1)
            copies(jnp.minimum(i0 + 2, D - 2), 0, True)
            wait_copies(i1, 1)
            compute(i1, 1, rr & 1)
            return carry

        lax.fori_loop(0, 4, pair, jnp.int32(0))
    wait_copies(jnp.int32(D - 2), 0)

    NB = SLOTS // 128
    for k in range(NB):
        for t in range(128 // L):
            b2[k, pl.ds(t * L, L)] = s2_b[pl.ds(k * 128 + t * L, L)]
            b2[k + NB, pl.ds(t * L, L)] = o2_b[pl.ds(k * 128 + t * L, L)]
    for k in range(NB):
        pltpu.sync_copy(acc_s.at[pl.ds(k * 128, 128)],
                        sh_s.at[b2.at[k]], add=True)
        pltpu.sync_copy(acc_ro.at[pl.ds(k * 128, 128)],
                        sh_ro.at[b2.at[k + NB]], add=True)
    plsc.subcore_barrier()
    sid = lax.axis_index("s")
    cid = lax.axis_index("c")

    @pl.when(sid < 8)
    def _writeback():
        off = sid * (B // 8)
        pltpu.sync_copy(sh_s.at[pl.ds(off, B // 8)],
                        out_s.at[cid, pl.ds(off, B // 8)])
        pltpu.sync_copy(sh_ro.at[pl.ds(off, B // 8)],
                        out_ro.at[cid, pl.ds(off, B // 8)])


def _mul_body(a0_ref, a1_ref, b0_ref, b1_ref, o_ref):
    o_ref[...] = ((a0_ref[...] + a1_ref[...])
                  * (b0_ref[...] + b1_ref[...]))


@jax.jit
def kernel(subjects, relations, objects, entity_table, relation_table):
    s = subjects.reshape(-1).astype(jnp.int32)
    r = relations.reshape(-1).astype(jnp.int32)
    o = objects.reshape(-1).astype(jnp.int32)
    ent_t = entity_table.T
    rel_flat = jnp.tile(jnp.pad(relation_table.T,
                                ((0, 0), (0, RELW - N_REL))).reshape(-1), NW)
    tail_flat = jnp.pad(entity_table[TAIL0:].T,
                        ((0, 0), (0, 128 - NTAIL))).reshape(-1)
    mesh = plsc.VectorSubcoreMesh(core_axis_name="c", subcore_axis_name="s")
    run = functools.partial(
        pl.kernel,
        mesh=mesh,
        compiler_params=pltpu.CompilerParams(needs_layout_passes=False),
        out_type=(jax.ShapeDtypeStruct((NC, B), jnp.float32),
                  jax.ShapeDtypeStruct((NC, B), jnp.float32)),
        scratch_types=[
            pltpu.VMEM((CHUNK,), jnp.int32),
            pltpu.VMEM((CHUNK,), jnp.int32),
            pltpu.VMEM((CHUNK,), jnp.int32),
            pltpu.VMEM((CHUNK,), jnp.int32),
            pltpu.VMEM((CHUNK,), jnp.int32),
            pltpu.VMEM((CHUNK,), jnp.int32),
            pltpu.VMEM((CAP,), jnp.int32),
            pltpu.VMEM((CAP,), jnp.int32),
            pltpu.VMEM((CAP,), jnp.int32),
            pltpu.VMEM((CAP,), jnp.int32),
            pltpu.VMEM((CAP,), jnp.int32),
            pltpu.VMEM((SLOTS,), jnp.int32),
            pltpu.VMEM((SLOTS,), jnp.int32),
            pltpu.VMEM((SLOTS,), jnp.int32),
            pltpu.VMEM((SLOTS,), jnp.int32),
            pltpu.VMEM((SLOTS,), jnp.int32),
            pltpu.VMEM((2 * SLOTS // 128, 128), jnp.int32),
            pltpu.SMEM((2 * NCHUNK,), jnp.int32),
            pltpu.VMEM((8, BW), jnp.float32),
            pltpu.VMEM((8, BW), jnp.float32),
            pltpu.VMEM((8 * RELW,), jnp.float32),
            pltpu.VMEM((8 * RELW,), jnp.float32),
            pltpu.VMEM((D * 128,), jnp.float32),
            pltpu.VMEM((SLOTS,), jnp.float32),
            pltpu.VMEM((SLOTS,), jnp.float32),
            pltpu.VMEM((4096,), jnp.float32),
            pltpu.VMEM_SHARED((B + 64,), jnp.float32),
            pltpu.VMEM_SHARED((B + 64,), jnp.float32),
            pltpu.SemaphoreType.DMA,
            pltpu.SemaphoreType.DMA,
            pltpu.SemaphoreType.DMA,
            pltpu.SemaphoreType.DMA,
            pltpu.SemaphoreType.DMA,
            pltpu.SemaphoreType.DMA,
            pltpu.SemaphoreType.DMA,
        ],
    )(_score_body)
    out_s, out_ro = run(s, r, o, ent_t, rel_flat, tail_flat)
    scores = pl.pallas_call(
        _mul_body,
        out_shape=jax.ShapeDtypeStruct((128, 128), jnp.float32),
    )(out_s[0].reshape(128, 128), out_s[1].reshape(128, 128),
      out_ro[0].reshape(128, 128), out_ro[1].reshape(128, 128))
    return scores.reshape(B, 1)


# speedup vs baseline: 3.3503x; 1.0159x over previous
import functools

import jax
import jax.numpy as jnp
from jax import lax
from jax.experimental import pallas as pl
from jax.experimental.pallas import tpu as pltpu
from jax.experimental.pallas import tpu_sc as plsc

B = 16384
D = 64
N_ENT = 1000000
N_REL = 1000
RELW = 1024
L = 16
NC = 2
NS = 16
NW = NC * NS
OWN = 31232
WBUF = 31744
TAIL0 = (NW - 1) * OWN + WBUF
NTAIL = N_ENT - TAIL0
CW = 3968
NCHUNK = WBUF // CW
BW = 4096
CAP = 1024
CCAP = 256
SLOTS = NCHUNK * CCAP
CHUNK = 1024


def _score_body(s_idx, r_idx, o_idx, ent_t, rel_flat, tail_flat,
                out_s, out_ro,
                idx_s, idx_o, idx_r, idx_s2, idx_o2, idx_r2,
                s_e, s_b, o_e, o_b, o_r,
                s2_e, s2_b, o2_e, o2_b, o2_r, b2, cnts,
                blk_a, blk_b, rel_a, rel_b, tailbuf, acc_s, acc_ro,
                zerobuf, sh_s, sh_ro,
                sem, sem_a, sem_b, sem_ra, sem_rb, sem_pa, sem_pb):
    wid = lax.axis_index("s") * NC + lax.axis_index("c")
    lo = wid * OWN
    hi = jnp.where(wid == NW - 1, N_ENT, lo + OWN)
    base = lo
    iota = lax.iota(jnp.int32, L)
    zero_i = jnp.zeros((L,), jnp.int32)
    zero_f = jnp.zeros((L,), jnp.float32)

    cp_tail = pltpu.async_copy(tail_flat, tailbuf, sem)

    def init1(v, carry):
        s_e[pl.ds(v * L, L)] = zero_i
        o_e[pl.ds(v * L, L)] = zero_i
        o_r[pl.ds(v * L, L)] = zero_i
        pad = B + ((v * L + iota) & 63)
        s_b[pl.ds(v * L, L)] = pad
        o_b[pl.ds(v * L, L)] = pad
        return carry

    lax.fori_loop(0, CAP // L, init1, jnp.int32(0))

    def init2(v, carry):
        s2_e[pl.ds(v * L, L)] = zero_i
        o2_e[pl.ds(v * L, L)] = zero_i
        o2_r[pl.ds(v * L, L)] = zero_i
        acc_s[pl.ds(v * L, L)] = zero_f
        acc_ro[pl.ds(v * L, L)] = zero_f
        pad = B + ((v * L + iota) & 63)
        s2_b[pl.ds(v * L, L)] = pad
        o2_b[pl.ds(v * L, L)] = pad
        return carry

    lax.fori_loop(0, SLOTS // L, init2, jnp.int32(0))

    def initz(v, carry):
        zerobuf[pl.ds(v * L, L)] = zero_f
        return carry

    lax.fori_loop(0, 4096 // L, initz, jnp.int32(0))

    @pl.when(lax.axis_index("s") == 0)
    def _zero_shared():
        for k in range(4):
            pltpu.sync_copy(zerobuf, sh_s.at[pl.ds(k * 4096, 4096)])
            pltpu.sync_copy(zerobuf, sh_ro.at[pl.ds(k * 4096, 4096)])
        pltpu.sync_copy(zerobuf.at[pl.ds(0, 64)], sh_s.at[pl.ds(B, 64)])
        pltpu.sync_copy(zerobuf.at[pl.ds(0, 64)], sh_ro.at[pl.ds(B, 64)])

    plsc.subcore_barrier()

    NP1 = B // CHUNK
    p1bufs = ((idx_s, idx_o, idx_r), (idx_s2, idx_o2, idx_r2))
    p1sems = (sem_pa, sem_pb)

    def p1stage(c, par, start):
        ctor = pltpu.async_copy if start else pltpu.make_async_copy
        bs, bo, br = p1bufs[par]
        hs = ctor(s_idx.at[pl.ds(c * CHUNK, CHUNK)], bs, p1sems[par])
        ho = ctor(o_idx.at[pl.ds(c * CHUNK, CHUNK)], bo, p1sems[par])
        hr = ctor(r_idx.at[pl.ds(c * CHUNK, CHUNK)], br, p1sems[par])
        return hs, ho, hr

    def p1wait(c, par):
        for h in p1stage(c, par, False):
            h.wait()

    def p1scan(c, par, cnts_io):
        bs, bo, br = p1bufs[par]

        def vec(v, cnts_io2):
            cs, co = cnts_io2
            bvec = c * CHUNK + v * L + iota
            e_s = bs[pl.ds(v * L, L)]
            e_o = bo[pl.ds(v * L, L)]
            m_s = (e_s >= lo) & (e_s < hi)
            m_o = (e_o >= lo) & (e_o < hi)
            n_s = plsc.all_reduce_population_count(m_s)[0]
            n_o = plsc.all_reduce_population_count(m_o)[0]
            plsc.store_compressed(s_e.at[pl.ds(cs, L)], e_s - base, mask=m_s)
            plsc.store_compressed(s_b.at[pl.ds(cs, L)], bvec, mask=m_s)
            plsc.store_compressed(o_e.at[pl.ds(co, L)], e_o - base, mask=m_o)
            plsc.store_compressed(o_b.at[pl.ds(co, L)], bvec, mask=m_o)
            rho = br[pl.ds(v * L, L)]
            plsc.store_compressed(o_r.at[pl.ds(co, L)], rho, mask=m_o)
            return (cs + n_s, co + n_o)

        return lax.fori_loop(0, CHUNK // L, vec, cnts_io)

    blk0_h = pltpu.async_copy(
        ent_t.at[pl.ds(0, 8), pl.ds(base + 0 * CW, CW)],
        blk_a.at[pl.ds(0, 8), pl.ds(0, CW)], sem_a)
    rel0_h = pltpu.async_copy(
        rel_flat.at[pl.ds(wid * (D * RELW), 8 * RELW)], rel_a, sem_ra)
    del blk0_h, rel0_h
    p1stage(jnp.int32(0), 0, True)

    def p1pair(j, cnts_io):
        c0 = 2 * j
        c1 = c0 + 1
        p1stage(c1, 1, True)
        p1wait(c0, 0)
        cnts_io = p1scan(c0, 0, cnts_io)
        p1stage(jnp.minimum(c0 + 2, NP1 - 2), 0, True)
        p1wait(c1, 1)
        cnts_io = p1scan(c1, 1, cnts_io)
        return cnts_io

    cnt_s, cnt_o = lax.fori_loop(0, NP1 // 2, p1pair,
                                 (jnp.int32(0), jnp.int32(0)))
    p1wait(jnp.int32(NP1 - 2), 0)

    nsv = (cnt_s + L - 1) >> 4
    nov = (cnt_o + L - 1) >> 4

    def rebucket(cc, carry):
        clo = cc * CW
        chi = jnp.where(cc == NCHUNK - 1, jnp.int32(2 ** 30), clo + CW)

        def rvec(v, cnts2):
            cs2, co2 = cnts2
            el_s = s_e[pl.ds(v * L, L)]
            el_o = o_e[pl.ds(v * L, L)]
            m_s = (el_s >= clo) & (el_s < chi) & (v < nsv)
            m_o = (el_o >= clo) & (el_o < chi) & (v < nov)
            n_s = plsc.all_reduce_population_count(m_s)[0]
            n_o = plsc.all_reduce_population_count(m_o)[0]
            plsc.store_compressed(
                s2_e.at[pl.ds(cc * CCAP + cs2, L)], el_s - clo, mask=m_s)
            plsc.store_compressed(
                s2_b.at[pl.ds(cc * CCAP + cs2, L)], s_b[pl.ds(v * L, L)],
                mask=m_s)
            plsc.store_compressed(
                o2_e.at[pl.ds(cc * CCAP + co2, L)], el_o - clo, mask=m_o)
            plsc.store_compressed(
                o2_b.at[pl.ds(cc * CCAP + co2, L)], o_b[pl.ds(v * L, L)],
                mask=m_o)
            plsc.store_compressed(
                o2_r.at[pl.ds(cc * CCAP + co2, L)], o_r[pl.ds(v * L, L)],
                mask=m_o)
            return (cs2 + n_s, co2 + n_o)

        cs_f, co_f = lax.fori_loop(0, jnp.maximum(nsv, nov), rvec,
                                   (jnp.int32(0), jnp.int32(0)))
        cnts[cc] = cs_f
        cnts[NCHUNK + cc] = co_f
        return carry

    lax.fori_loop(0, NCHUNK, rebucket, jnp.int32(0))
    cp_tail.wait()

    blks = (blk_a, blk_b)
    rels = (rel_a, rel_b)
    sems = (sem_a, sem_b)

    rsems = (sem_ra, sem_rb)

    def rel_copy(rr, start):
        ctor = pltpu.async_copy if start else pltpu.make_async_copy
        return ctor(
            rel_flat.at[pl.ds(wid * (D * RELW) + rr * (8 * RELW), 8 * RELW)],
            rels[rr % 2], rsems[rr % 2])

    def copies(i, p, start):
        rr = i >> 3
        cc = i & 7
        ctor = pltpu.async_copy if start else pltpu.make_async_copy
        h = ctor(ent_t.at[pl.ds(rr * 8, 8), pl.ds(base + cc * CW, CW)],
                 blks[p].at[pl.ds(0, 8), pl.ds(0, CW)], sems[p])
        return h

    def wait_copies(i, p):
        copies(i, p, False).wait()

    def compute(i, p, rp):
        rr = i >> 3
        cc = i & 7
        blk = blks[p]
        rel = rels[rp]

        @pl.when(cc == NCHUNK - 1)
        def _append_tail():
            def tmove(dd, carry):
                for t in range(NTAIL // L):
                    blk[dd, pl.ds(CW + t * L, L)] = (
                        tailbuf[pl.ds((rr * 8 + dd) * 128 + t * L, L)])
                return carry

            lax.fori_loop(0, 8, tmove, jnp.int32(0))

        nscv = (cnts[cc] + L - 1) >> 4
        nocv = (cnts[NCHUNK + cc] + L - 1) >> 4

        def sv(v, carry2):
            el = s2_e[pl.ds(cc * CCAP + v * L, L)]
            t = plsc.load_gather(blk, [iota * 0, el])
            for dd in range(1, 8):
                t = t + plsc.load_gather(blk, [iota * 0 + dd, el])
            plsc.addupdate(acc_s.at[pl.ds(cc * CCAP + v * L, L)], t)
            return carry2

        lax.fori_loop(0, nscv, sv, jnp.int32(0))

        def ov(v, carry2):
            el = o2_e[pl.ds(cc * CCAP + v * L, L)]
            rho = o2_r[pl.ds(cc * CCAP + v * L, L)]
            t = (plsc.load_gather(blk, [iota * 0, el])
                 * plsc.load_gather(rel, [rho]))
            for dd in range(1, 8):
                t = t + (plsc.load_gather(blk, [iota * 0 + dd, el])
                         * plsc.load_gather(rel, [rho + dd * RELW]))
            plsc.addupdate(acc_ro.at[pl.ds(cc * CCAP + v * L, L)], t)
            return carry2

        lax.fori_loop(0, nocv, ov, jnp.int32(0))

    for rr in range(8):
        if rr + 1 < 8:
            rel_copy(rr + 1, True)
        rel_copy(rr, False).wait()

        def pair(j, carry, rr=rr):
            i0 = rr * 8 + 2 * j
            i1 = i0 + 1
            copies(i1, 1, True)
            wait_copies(i0, 0)
            compute(i0, 0, rr & 1)
            copies(jnp.minimum(i0 + 2, D - 2), 0, True)
            wait_copies(i1, 1)
            compute(i1, 1, rr & 1)
            return carry

        lax.fori_loop(0, 4, pair, jnp.int32(0))
    wait_copies(jnp.int32(D - 2), 0)

    NB = SLOTS // 128
    for k in range(NB):
        for t in range(128 // L):
            b2[k, pl.ds(t * L, L)] = s2_b[pl.ds(k * 128 + t * L, L)]
            b2[k + NB, pl.ds(t * L, L)] = o2_b[pl.ds(k * 128 + t * L, L)]
    swaits = []
    for k in range(NB):
        swaits.append(pltpu.async_copy(
            acc_s.at[pl.ds(k * 128, 128)], sh_s.at[b2.at[k]], sem,
            add=True))
        swaits.append(pltpu.async_copy(
            acc_ro.at[pl.ds(k * 128, 128)], sh_ro.at[b2.at[k + NB]], sem,
            add=True))
    for w in swaits:
        w.wait()
    plsc.subcore_barrier()
    sid = lax.axis_index("s")
    cid = lax.axis_index("c")

    @pl.when(sid < 8)
    def _writeback():
        off = sid * (B // 8)
        pltpu.sync_copy(sh_s.at[pl.ds(off, B // 8)],
                        out_s.at[cid, pl.ds(off, B // 8)])
        pltpu.sync_copy(sh_ro.at[pl.ds(off, B // 8)],
                        out_ro.at[cid, pl.ds(off, B // 8)])


def _mul_body(a0_ref, a1_ref, b0_ref, b1_ref, o_ref):
    o_ref[...] = ((a0_ref[...] + a1_ref[...])
                  * (b0_ref[...] + b1_ref[...]))


@jax.jit
def kernel(subjects, relations, objects, entity_table, relation_table):
    s = subjects.reshape(-1).astype(jnp.int32)
    r = relations.reshape(-1).astype(jnp.int32)
    o = objects.reshape(-1).astype(jnp.int32)
    ent_t = entity_table.T
    rel_flat = jnp.tile(jnp.pad(relation_table.T,
                                ((0, 0), (0, RELW - N_REL))).reshape(-1), NW)
    tail_flat = jnp.pad(entity_table[TAIL0:].T,
                        ((0, 0), (0, 128 - NTAIL))).reshape(-1)
    mesh = plsc.VectorSubcoreMesh(core_axis_name="c", subcore_axis_name="s")
    run = functools.partial(
        pl.kernel,
        mesh=mesh,
        compiler_params=pltpu.CompilerParams(needs_layout_passes=False),
        out_type=(jax.ShapeDtypeStruct((NC, B), jnp.float32),
                  jax.ShapeDtypeStruct((NC, B), jnp.float32)),
        scratch_types=[
            pltpu.VMEM((CHUNK,), jnp.int32),
            pltpu.VMEM((CHUNK,), jnp.int32),
            pltpu.VMEM((CHUNK,), jnp.int32),
            pltpu.VMEM((CHUNK,), jnp.int32),
            pltpu.VMEM((CHUNK,), jnp.int32),
            pltpu.VMEM((CHUNK,), jnp.int32),
            pltpu.VMEM((CAP,), jnp.int32),
            pltpu.VMEM((CAP,), jnp.int32),
            pltpu.VMEM((CAP,), jnp.int32),
            pltpu.VMEM((CAP,), jnp.int32),
            pltpu.VMEM((CAP,), jnp.int32),
            pltpu.VMEM((SLOTS,), jnp.int32),
            pltpu.VMEM((SLOTS,), jnp.int32),
            pltpu.VMEM((SLOTS,), jnp.int32),
            pltpu.VMEM((SLOTS,), jnp.int32),
            pltpu.VMEM((SLOTS,), jnp.int32),
            pltpu.VMEM((2 * SLOTS // 128, 128), jnp.int32),
            pltpu.SMEM((2 * NCHUNK,), jnp.int32),
            pltpu.VMEM((8, BW), jnp.float32),
            pltpu.VMEM((8, BW), jnp.float32),
            pltpu.VMEM((8 * RELW,), jnp.float32),
            pltpu.VMEM((8 * RELW,), jnp.float32),
            pltpu.VMEM((D * 128,), jnp.float32),
            pltpu.VMEM((SLOTS,), jnp.float32),
            pltpu.VMEM((SLOTS,), jnp.float32),
            pltpu.VMEM((4096,), jnp.float32),
            pltpu.VMEM_SHARED((B + 64,), jnp.float32),
            pltpu.VMEM_SHARED((B + 64,), jnp.float32),
            pltpu.SemaphoreType.DMA,
            pltpu.SemaphoreType.DMA,
            pltpu.SemaphoreType.DMA,
            pltpu.SemaphoreType.DMA,
            pltpu.SemaphoreType.DMA,
            pltpu.SemaphoreType.DMA,
            pltpu.SemaphoreType.DMA,
        ],
    )(_score_body)
    out_s, out_ro = run(s, r, o, ent_t, rel_flat, tail_flat)
    scores = pl.pallas_call(
        _mul_body,
        out_shape=jax.ShapeDtypeStruct((128, 128), jnp.float32),
    )(out_s[0].reshape(128, 128), out_s[1].reshape(128, 128),
      out_ro[0].reshape(128, 128), out_ro[1].reshape(128, 128))
    return scores.reshape(B, 1)


# speedup vs baseline: 3.6577x; 1.0918x over previous
import functools

import jax
import jax.numpy as jnp
from jax import lax
from jax.experimental import pallas as pl
from jax.experimental.pallas import tpu as pltpu
from jax.experimental.pallas import tpu_sc as plsc

B = 16384
D = 64
N_ENT = 1000000
N_REL = 1000
RELW = 1024
L = 16
NC = 2
NS = 16
NW = NC * NS
OWN = 31232
WBUF = 31744
TAIL0 = (NW - 1) * OWN + WBUF
NTAIL = N_ENT - TAIL0
CW = 3968
NCHUNK = WBUF // CW
BW = 4096
CAP = 1024
CCAP = 256
SLOTS = NCHUNK * CCAP
CHUNK = 1024


def _score_body(s_idx, r_idx, o_idx, ent_t, rel_flat, tail_flat,
                out_s, out_ro,
                idx_s, idx_o, idx_r, idx_s2, idx_o2, idx_r2,
                s_e, s_b, o_e, o_b, o_r,
                s2_e, s2_b, o2_e, o2_b, o2_r, b2, cnts,
                blk_a, blk_b, rel_a, rel_b, tailbuf, acc_s, acc_ro,
                zerobuf, sh_s, sh_ro,
                sem, sem_a, sem_b, sem_ra, sem_rb, sem_pa, sem_pb):
    wid = lax.axis_index("s") * NC + lax.axis_index("c")
    lo = wid * OWN
    hi = jnp.where(wid == NW - 1, N_ENT, lo + OWN)
    base = lo
    iota = lax.iota(jnp.int32, L)
    zero_i = jnp.zeros((L,), jnp.int32)
    zero_f = jnp.zeros((L,), jnp.float32)

    cp_tail = pltpu.async_copy(tail_flat, tailbuf, sem)

    def init1(v, carry):
        s_e[pl.ds(v * L, L)] = zero_i
        o_e[pl.ds(v * L, L)] = zero_i
        o_r[pl.ds(v * L, L)] = zero_i
        pad = B + ((v * L + iota) & 63)
        s_b[pl.ds(v * L, L)] = pad
        o_b[pl.ds(v * L, L)] = pad
        return carry

    lax.fori_loop(0, CAP // L, init1, jnp.int32(0))

    def init2(v, carry):
        s2_e[pl.ds(v * L, L)] = zero_i
        o2_e[pl.ds(v * L, L)] = zero_i
        o2_r[pl.ds(v * L, L)] = zero_i
        acc_s[pl.ds(v * L, L)] = zero_f
        acc_ro[pl.ds(v * L, L)] = zero_f
        pad = B + ((v * L + iota) & 63)
        s2_b[pl.ds(v * L, L)] = pad
        o2_b[pl.ds(v * L, L)] = pad
        return carry

    lax.fori_loop(0, SLOTS // L, init2, jnp.int32(0))

    def initz(v, carry):
        zerobuf[pl.ds(v * L, L)] = zero_f
        return carry

    lax.fori_loop(0, 4096 // L, initz, jnp.int32(0))

    @pl.when(lax.axis_index("s") == 0)
    def _zero_shared():
        for k in range(4):
            pltpu.sync_copy(zerobuf, sh_s.at[pl.ds(k * 4096, 4096)])
            pltpu.sync_copy(zerobuf, sh_ro.at[pl.ds(k * 4096, 4096)])
        pltpu.sync_copy(zerobuf.at[pl.ds(0, 64)], sh_s.at[pl.ds(B, 64)])
        pltpu.sync_copy(zerobuf.at[pl.ds(0, 64)], sh_ro.at[pl.ds(B, 64)])

    plsc.subcore_barrier()

    NP1 = B // CHUNK
    p1bufs = ((idx_s, idx_o, idx_r), (idx_s2, idx_o2, idx_r2))
    p1sems = (sem_pa, sem_pb)

    def p1stage(c, par, start):
        ctor = pltpu.async_copy if start else pltpu.make_async_copy
        bs, bo, br = p1bufs[par]
        hs = ctor(s_idx.at[pl.ds(c * CHUNK, CHUNK)], bs, p1sems[par])
        ho = ctor(o_idx.at[pl.ds(c * CHUNK, CHUNK)], bo, p1sems[par])
        hr = ctor(r_idx.at[pl.ds(c * CHUNK, CHUNK)], br, p1sems[par])
        return hs, ho, hr

    def p1wait(c, par):
        for h in p1stage(c, par, False):
            h.wait()

    def p1scan(c, par, cnts_io):
        bs, bo, br = p1bufs[par]

        def vec(v, cnts_io2):
            cs, co = cnts_io2
            bvec = c * CHUNK + v * L + iota
            e_s = bs[pl.ds(v * L, L)]
            e_o = bo[pl.ds(v * L, L)]
            m_s = (e_s >= lo) & (e_s < hi)
            m_o = (e_o >= lo) & (e_o < hi)
            n_s = plsc.all_reduce_population_count(m_s)[0]
            n_o = plsc.all_reduce_population_count(m_o)[0]
            plsc.store_compressed(s_e.at[pl.ds(cs, L)], e_s - base, mask=m_s)
            plsc.store_compressed(s_b.at[pl.ds(cs, L)], bvec, mask=m_s)
            plsc.store_compressed(o_e.at[pl.ds(co, L)], e_o - base, mask=m_o)
            plsc.store_compressed(o_b.at[pl.ds(co, L)], bvec, mask=m_o)
            rho = br[pl.ds(v * L, L)]
            plsc.store_compressed(o_r.at[pl.ds(co, L)], rho, mask=m_o)
            return (cs + n_s, co + n_o)

        return lax.fori_loop(0, CHUNK // L, vec, cnts_io)

    blk0_h = pltpu.async_copy(
        ent_t.at[pl.ds(0, 8), pl.ds(base + 0 * CW, CW)],
        blk_a.at[pl.ds(0, 8), pl.ds(0, CW)], sem_a)
    rel0_h = pltpu.async_copy(
        rel_flat.at[pl.ds((wid >> 2) * (D * RELW), 8 * RELW)], rel_a, sem_ra)
    del blk0_h, rel0_h
    p1stage(jnp.int32(0), 0, True)

    def p1pair(j, cnts_io):
        c0 = 2 * j
        c1 = c0 + 1
        p1stage(c1, 1, True)
        p1wait(c0, 0)
        cnts_io = p1scan(c0, 0, cnts_io)
        p1stage(jnp.minimum(c0 + 2, NP1 - 2), 0, True)
        p1wait(c1, 1)
        cnts_io = p1scan(c1, 1, cnts_io)
        return cnts_io

    cnt_s, cnt_o = lax.fori_loop(0, NP1 // 2, p1pair,
                                 (jnp.int32(0), jnp.int32(0)))
    p1wait(jnp.int32(NP1 - 2), 0)

    nsv = (cnt_s + L - 1) >> 4
    nov = (cnt_o + L - 1) >> 4

    def rebucket(cc, carry):
        clo = cc * CW
        chi = jnp.where(cc == NCHUNK - 1, jnp.int32(2 ** 30), clo + CW)

        def rvec(v, cnts2):
            cs2, co2 = cnts2
            el_s = s_e[pl.ds(v * L, L)]
            el_o = o_e[pl.ds(v * L, L)]
            m_s = (el_s >= clo) & (el_s < chi) & (v < nsv)
            m_o = (el_o >= clo) & (el_o < chi) & (v < nov)
            n_s = plsc.all_reduce_population_count(m_s)[0]
            n_o = plsc.all_reduce_population_count(m_o)[0]
            plsc.store_compressed(
                s2_e.at[pl.ds(cc * CCAP + cs2, L)], el_s - clo, mask=m_s)
            plsc.store_compressed(
                s2_b.at[pl.ds(cc * CCAP + cs2, L)], s_b[pl.ds(v * L, L)],
                mask=m_s)
            plsc.store_compressed(
                o2_e.at[pl.ds(cc * CCAP + co2, L)], el_o - clo, mask=m_o)
            plsc.store_compressed(
                o2_b.at[pl.ds(cc * CCAP + co2, L)], o_b[pl.ds(v * L, L)],
                mask=m_o)
            plsc.store_compressed(
                o2_r.at[pl.ds(cc * CCAP + co2, L)], o_r[pl.ds(v * L, L)],
                mask=m_o)
            return (cs2 + n_s, co2 + n_o)

        cs_f, co_f = lax.fori_loop(0, jnp.maximum(nsv, nov), rvec,
                                   (jnp.int32(0), jnp.int32(0)))
        cnts[cc] = cs_f
        cnts[NCHUNK + cc] = co_f
        return carry

    lax.fori_loop(0, NCHUNK, rebucket, jnp.int32(0))
    cp_tail.wait()

    blks = (blk_a, blk_b)
    rels = (rel_a, rel_b)
    sems = (sem_a, sem_b)

    rsems = (sem_ra, sem_rb)

    def rel_copy(rr, start):
        ctor = pltpu.async_copy if start else pltpu.make_async_copy
        return ctor(
            rel_flat.at[pl.ds((wid >> 2) * (D * RELW) + rr * (8 * RELW), 8 * RELW)],
            rels[rr % 2], rsems[rr % 2])

    def copies(i, p, start):
        rr = i >> 3
        cc = i & 7
        ctor = pltpu.async_copy if start else pltpu.make_async_copy
        h = ctor(ent_t.at[pl.ds(rr * 8, 8), pl.ds(base + cc * CW, CW)],
                 blks[p].at[pl.ds(0, 8), pl.ds(0, CW)], sems[p])
        return h

    def wait_copies(i, p):
        copies(i, p, False).wait()

    def compute(i, p, rp):
        rr = i >> 3
        cc = i & 7
        blk = blks[p]
        rel = rels[rp]

        @pl.when(cc == NCHUNK - 1)
        def _append_tail():
            def tmove(dd, carry):
                for t in range(NTAIL // L):
                    blk[dd, pl.ds(CW + t * L, L)] = (
                        tailbuf[pl.ds((rr * 8 + dd) * 128 + t * L, L)])
                return carry

            lax.fori_loop(0, 8, tmove, jnp.int32(0))

        nscv = (cnts[cc] + L - 1) >> 4
        nocv = (cnts[NCHUNK + cc] + L - 1) >> 4

        def sv(v, carry2):
            el = s2_e[pl.ds(cc * CCAP + v * L, L)]
            t = plsc.load_gather(blk, [iota * 0, el])
            for dd in range(1, 8):
                t = t + plsc.load_gather(blk, [iota * 0 + dd, el])
            plsc.addupdate(acc_s.at[pl.ds(cc * CCAP + v * L, L)], t)
            return carry2

        lax.fori_loop(0, nscv, sv, jnp.int32(0))

        def ov(v, carry2):
            el = o2_e[pl.ds(cc * CCAP + v * L, L)]
            rho = o2_r[pl.ds(cc * CCAP + v * L, L)]
            t = (plsc.load_gather(blk, [iota * 0, el])
                 * plsc.load_gather(rel, [rho]))
            for dd in range(1, 8):
                t = t + (plsc.load_gather(blk, [iota * 0 + dd, el])
                         * plsc.load_gather(rel, [rho + dd * RELW]))
            plsc.addupdate(acc_ro.at[pl.ds(cc * CCAP + v * L, L)], t)
            return carry2

        lax.fori_loop(0, nocv, ov, jnp.int32(0))

    for rr in range(8):
        if rr + 1 < 8:
            rel_copy(rr + 1, True)
        rel_copy(rr, False).wait()

        def pair(j, carry, rr=rr):
            i0 = rr * 8 + 2 * j
            i1 = i0 + 1
            copies(i1, 1, True)
            wait_copies(i0, 0)
            compute(i0, 0, rr & 1)
            copies(jnp.minimum(i0 + 2, D - 2), 0, True)
            wait_copies(i1, 1)
            compute(i1, 1, rr & 1)
            return carry

        lax.fori_loop(0, 4, pair, jnp.int32(0))
    wait_copies(jnp.int32(D - 2), 0)

    NB = SLOTS // 128
    for k in range(NB):
        for t in range(128 // L):
            b2[k, pl.ds(t * L, L)] = s2_b[pl.ds(k * 128 + t * L, L)]
            b2[k + NB, pl.ds(t * L, L)] = o2_b[pl.ds(k * 128 + t * L, L)]
    swaits = []
    for k in range(NB):
        swaits.append(pltpu.async_copy(
            acc_s.at[pl.ds(k * 128, 128)], sh_s.at[b2.at[k]], sem,
            add=True))
        swaits.append(pltpu.async_copy(
            acc_ro.at[pl.ds(k * 128, 128)], sh_ro.at[b2.at[k + NB]], sem,
            add=True))
    for w in swaits:
        w.wait()
    plsc.subcore_barrier()
    sid = lax.axis_index("s")
    cid = lax.axis_index("c")

    @pl.when(sid < 8)
    def _writeback():
        off = sid * (B // 8)
        pltpu.sync_copy(sh_s.at[pl.ds(off, B // 8)],
                        out_s.at[cid, pl.ds(off, B // 8)])
        pltpu.sync_copy(sh_ro.at[pl.ds(off, B // 8)],
                        out_ro.at[cid, pl.ds(off, B // 8)])


def _mul_body(a0_ref, a1_ref, b0_ref, b1_ref, o_ref):
    o_ref[...] = ((a0_ref[...] + a1_ref[...])
                  * (b0_ref[...] + b1_ref[...]))


@jax.jit
def kernel(subjects, relations, objects, entity_table, relation_table):
    s = subjects.reshape(-1).astype(jnp.int32)
    r = relations.reshape(-1).astype(jnp.int32)
    o = objects.reshape(-1).astype(jnp.int32)
    ent_t = entity_table.T
    rel_flat = jnp.tile(jnp.pad(relation_table.T,
                                ((0, 0), (0, RELW - N_REL))).reshape(-1), NW // 4)
    tail_flat = jnp.pad(entity_table[TAIL0:].T,
                        ((0, 0), (0, 128 - NTAIL))).reshape(-1)
    mesh = plsc.VectorSubcoreMesh(core_axis_name="c", subcore_axis_name="s")
    run = functools.partial(
        pl.kernel,
        mesh=mesh,
        compiler_params=pltpu.CompilerParams(needs_layout_passes=False),
        out_type=(jax.ShapeDtypeStruct((NC, B), jnp.float32),
                  jax.ShapeDtypeStruct((NC, B), jnp.float32)),
        scratch_types=[
            pltpu.VMEM((CHUNK,), jnp.int32),
            pltpu.VMEM((CHUNK,), jnp.int32),
            pltpu.VMEM((CHUNK,), jnp.int32),
            pltpu.VMEM((CHUNK,), jnp.int32),
            pltpu.VMEM((CHUNK,), jnp.int32),
            pltpu.VMEM((CHUNK,), jnp.int32),
            pltpu.VMEM((CAP,), jnp.int32),
            pltpu.VMEM((CAP,), jnp.int32),
            pltpu.VMEM((CAP,), jnp.int32),
            pltpu.VMEM((CAP,), jnp.int32),
            pltpu.VMEM((CAP,), jnp.int32),
            pltpu.VMEM((SLOTS,), jnp.int32),
            pltpu.VMEM((SLOTS,), jnp.int32),
            pltpu.VMEM((SLOTS,), jnp.int32),
            pltpu.VMEM((SLOTS,), jnp.int32),
            pltpu.VMEM((SLOTS,), jnp.int32),
            pltpu.VMEM((2 * SLOTS // 128, 128), jnp.int32),
            pltpu.SMEM((2 * NCHUNK,), jnp.int32),
            pltpu.VMEM((8, BW), jnp.float32),
            pltpu.VMEM((8, BW), jnp.float32),
            pltpu.VMEM((8 * RELW,), jnp.float32),
            pltpu.VMEM((8 * RELW,), jnp.float32),
            pltpu.VMEM((D * 128,), jnp.float32),
            pltpu.VMEM((SLOTS,), jnp.float32),
            pltpu.VMEM((SLOTS,), jnp.float32),
            pltpu.VMEM((4096,), jnp.float32),
            pltpu.VMEM_SHARED((B + 64,), jnp.float32),
            pltpu.VMEM_SHARED((B + 64,), jnp.float32),
            pltpu.SemaphoreType.DMA,
            pltpu.SemaphoreType.DMA,
            pltpu.SemaphoreType.DMA,
            pltpu.SemaphoreType.DMA,
            pltpu.SemaphoreType.DMA,
            pltpu.SemaphoreType.DMA,
            pltpu.SemaphoreType.DMA,
        ],
    )(_score_body)
    out_s, out_ro = run(s, r, o, ent_t, rel_flat, tail_flat)
    scores = pl.pallas_call(
        _mul_body,
        out_shape=jax.ShapeDtypeStruct((128, 128), jnp.float32),
    )(out_s[0].reshape(128, 128), out_s[1].reshape(128, 128),
      out_ro[0].reshape(128, 128), out_ro[1].reshape(128, 128))
    return scores.reshape(B, 1)


# speedup vs baseline: 3.6813x; 1.0065x over previous
import functools

import jax
import jax.numpy as jnp
from jax import lax
from jax.experimental import pallas as pl
from jax.experimental.pallas import tpu as pltpu
from jax.experimental.pallas import tpu_sc as plsc

B = 16384
D = 64
N_ENT = 1000000
N_REL = 1000
RELW = 1024
L = 16
NC = 2
NS = 16
NW = NC * NS
OWN = 31232
WBUF = 31744
TAIL0 = (NW - 1) * OWN + WBUF
NTAIL = N_ENT - TAIL0
CW = 3968
NCHUNK = WBUF // CW
BW = 4096
CAP = 1024
CCAP = 256
SLOTS = NCHUNK * CCAP
CHUNK = 1024


def _score_body(s_idx, r_idx, o_idx, ent_t, rel_flat, tail_flat,
                out_s, out_ro,
                idx_s, idx_o, idx_r, idx_s2, idx_o2, idx_r2,
                s_e, s_b, o_e, o_b, o_r,
                s2_e, s2_b, o2_e, o2_b, o2_r, b2, cnts,
                blk_a, blk_b, rel_a, rel_b, tailbuf, acc_s, acc_ro,
                zerobuf, sh_s, sh_ro,
                sem, sem_a, sem_b, sem_ra, sem_rb, sem_pa, sem_pb):
    wid = lax.axis_index("s") * NC + lax.axis_index("c")
    lo = wid * OWN
    hi = jnp.where(wid == NW - 1, N_ENT, lo + OWN)
    base = lo
    iota = lax.iota(jnp.int32, L)
    zero_i = jnp.zeros((L,), jnp.int32)
    zero_f = jnp.zeros((L,), jnp.float32)

    cp_tail = pltpu.async_copy(tail_flat, tailbuf, sem)

    def init1(v, carry):
        s_e[pl.ds(v * L, L)] = zero_i
        o_e[pl.ds(v * L, L)] = zero_i
        o_r[pl.ds(v * L, L)] = zero_i
        pad = B + ((v * L + iota) & 63)
        s_b[pl.ds(v * L, L)] = pad
        o_b[pl.ds(v * L, L)] = pad
        return carry

    lax.fori_loop(0, CAP // L, init1, jnp.int32(0))

    def init2(v, carry):
        s2_e[pl.ds(v * L, L)] = zero_i
        o2_e[pl.ds(v * L, L)] = zero_i
        o2_r[pl.ds(v * L, L)] = zero_i
        acc_s[pl.ds(v * L, L)] = zero_f
        acc_ro[pl.ds(v * L, L)] = zero_f
        pad = B + ((v * L + iota) & 63)
        s2_b[pl.ds(v * L, L)] = pad
        o2_b[pl.ds(v * L, L)] = pad
        return carry

    lax.fori_loop(0, SLOTS // L, init2, jnp.int32(0))

    def initz(v, carry):
        zerobuf[pl.ds(v * L, L)] = zero_f
        return carry

    lax.fori_loop(0, 4096 // L, initz, jnp.int32(0))

    @pl.when(lax.axis_index("s") == 0)
    def _zero_shared():
        for k in range(4):
            pltpu.sync_copy(zerobuf, sh_s.at[pl.ds(k * 4096, 4096)])
            pltpu.sync_copy(zerobuf, sh_ro.at[pl.ds(k * 4096, 4096)])
        pltpu.sync_copy(zerobuf.at[pl.ds(0, 64)], sh_s.at[pl.ds(B, 64)])
        pltpu.sync_copy(zerobuf.at[pl.ds(0, 64)], sh_ro.at[pl.ds(B, 64)])

    plsc.subcore_barrier()

    NP1 = B // CHUNK
    p1bufs = ((idx_s, idx_o, idx_r), (idx_s2, idx_o2, idx_r2))
    p1sems = (sem_pa, sem_pb)

    def p1stage(c, par, start):
        ctor = pltpu.async_copy if start else pltpu.make_async_copy
        bs, bo, br = p1bufs[par]
        hs = ctor(s_idx.at[pl.ds(c * CHUNK, CHUNK)], bs, p1sems[par])
        ho = ctor(o_idx.at[pl.ds(c * CHUNK, CHUNK)], bo, p1sems[par])
        hr = ctor(r_idx.at[pl.ds(c * CHUNK, CHUNK)], br, p1sems[par])
        return hs, ho, hr

    def p1wait(c, par):
        for h in p1stage(c, par, False):
            h.wait()

    def p1scan(c, par, cnts_io):
        bs, bo, br = p1bufs[par]

        def vec(v, cnts_io2):
            cs, co = cnts_io2
            bvec = c * CHUNK + v * L + iota
            e_s = bs[pl.ds(v * L, L)]
            e_o = bo[pl.ds(v * L, L)]
            m_s = (e_s >= lo) & (e_s < hi)
            m_o = (e_o >= lo) & (e_o < hi)
            n_s = plsc.all_reduce_population_count(m_s)[0]
            n_o = plsc.all_reduce_population_count(m_o)[0]
            plsc.store_compressed(s_e.at[pl.ds(cs, L)], e_s - base, mask=m_s)
            plsc.store_compressed(s_b.at[pl.ds(cs, L)], bvec, mask=m_s)
            plsc.store_compressed(o_e.at[pl.ds(co, L)], e_o - base, mask=m_o)
            plsc.store_compressed(o_b.at[pl.ds(co, L)], bvec, mask=m_o)
            rho = br[pl.ds(v * L, L)]
            plsc.store_compressed(o_r.at[pl.ds(co, L)], rho, mask=m_o)
            return (cs + n_s, co + n_o)

        return lax.fori_loop(0, CHUNK // L, vec, cnts_io)

    blk0_h = pltpu.async_copy(
        ent_t.at[pl.ds(0, 8), pl.ds(base + 0 * CW, CW)],
        blk_a.at[pl.ds(0, 8), pl.ds(0, CW)], sem_a)
    rel0_h = pltpu.async_copy(
        rel_flat.at[pl.ds((wid >> 3) * (D * RELW), 8 * RELW)], rel_a, sem_ra)
    del blk0_h, rel0_h
    p1stage(jnp.int32(0), 0, True)

    def p1pair(j, cnts_io):
        c0 = 2 * j
        c1 = c0 + 1
        p1stage(c1, 1, True)
        p1wait(c0, 0)
        cnts_io = p1scan(c0, 0, cnts_io)
        p1stage(jnp.minimum(c0 + 2, NP1 - 2), 0, True)
        p1wait(c1, 1)
        cnts_io = p1scan(c1, 1, cnts_io)
        return cnts_io

    cnt_s, cnt_o = lax.fori_loop(0, NP1 // 2, p1pair,
                                 (jnp.int32(0), jnp.int32(0)))
    p1wait(jnp.int32(NP1 - 2), 0)

    nsv = (cnt_s + L - 1) >> 4
    nov = (cnt_o + L - 1) >> 4

    def rebucket(cc, carry):
        clo = cc * CW
        chi = jnp.where(cc == NCHUNK - 1, jnp.int32(2 ** 30), clo + CW)

        def rvec(v, cnts2):
            cs2, co2 = cnts2
            el_s = s_e[pl.ds(v * L, L)]
            el_o = o_e[pl.ds(v * L, L)]
            m_s = (el_s >= clo) & (el_s < chi) & (v < nsv)
            m_o = (el_o >= clo) & (el_o < chi) & (v < nov)
            n_s = plsc.all_reduce_population_count(m_s)[0]
            n_o = plsc.all_reduce_population_count(m_o)[0]
            plsc.store_compressed(
                s2_e.at[pl.ds(cc * CCAP + cs2, L)], el_s - clo, mask=m_s)
            plsc.store_compressed(
                s2_b.at[pl.ds(cc * CCAP + cs2, L)], s_b[pl.ds(v * L, L)],
                mask=m_s)
            plsc.store_compressed(
                o2_e.at[pl.ds(cc * CCAP + co2, L)], el_o - clo, mask=m_o)
            plsc.store_compressed(
                o2_b.at[pl.ds(cc * CCAP + co2, L)], o_b[pl.ds(v * L, L)],
                mask=m_o)
            plsc.store_compressed(
                o2_r.at[pl.ds(cc * CCAP + co2, L)], o_r[pl.ds(v * L, L)],
                mask=m_o)
            return (cs2 + n_s, co2 + n_o)

        cs_f, co_f = lax.fori_loop(0, jnp.maximum(nsv, nov), rvec,
                                   (jnp.int32(0), jnp.int32(0)))
        cnts[cc] = cs_f
        cnts[NCHUNK + cc] = co_f
        return carry

    lax.fori_loop(0, NCHUNK, rebucket, jnp.int32(0))
    cp_tail.wait()

    blks = (blk_a, blk_b)
    rels = (rel_a, rel_b)
    sems = (sem_a, sem_b)

    rsems = (sem_ra, sem_rb)

    def rel_copy(rr, start):
        ctor = pltpu.async_copy if start else pltpu.make_async_copy
        return ctor(
            rel_flat.at[pl.ds((wid >> 3) * (D * RELW) + rr * (8 * RELW), 8 * RELW)],
            rels[rr % 2], rsems[rr % 2])

    def copies(i, p, start):
        rr = i >> 3
        cc = i & 7
        ctor = pltpu.async_copy if start else pltpu.make_async_copy
        h = ctor(ent_t.at[pl.ds(rr * 8, 8), pl.ds(base + cc * CW, CW)],
                 blks[p].at[pl.ds(0, 8), pl.ds(0, CW)], sems[p])
        return h

    def wait_copies(i, p):
        copies(i, p, False).wait()

    def compute(i, p, rp):
        rr = i >> 3
        cc = i & 7
        blk = blks[p]
        rel = rels[rp]

        @pl.when(cc == NCHUNK - 1)
        def _append_tail():
            def tmove(dd, carry):
                for t in range(NTAIL // L):
                    blk[dd, pl.ds(CW + t * L, L)] = (
                        tailbuf[pl.ds((rr * 8 + dd) * 128 + t * L, L)])
                return carry

            lax.fori_loop(0, 8, tmove, jnp.int32(0))

        nscv = (cnts[cc] + L - 1) >> 4
        nocv = (cnts[NCHUNK + cc] + L - 1) >> 4

        def sv(v, carry2):
            el = s2_e[pl.ds(cc * CCAP + v * L, L)]
            t = plsc.load_gather(blk, [iota * 0, el])
            for dd in range(1, 8):
                t = t + plsc.load_gather(blk, [iota * 0 + dd, el])
            plsc.addupdate(acc_s.at[pl.ds(cc * CCAP + v * L, L)], t)
            return carry2

        lax.fori_loop(0, nscv, sv, jnp.int32(0))

        def ov(v, carry2):
            el = o2_e[pl.ds(cc * CCAP + v * L, L)]
            rho = o2_r[pl.ds(cc * CCAP + v * L, L)]
            t = (plsc.load_gather(blk, [iota * 0, el])
                 * plsc.load_gather(rel, [rho]))
            for dd in range(1, 8):
                t = t + (plsc.load_gather(blk, [iota * 0 + dd, el])
                         * plsc.load_gather(rel, [rho + dd * RELW]))
            plsc.addupdate(acc_ro.at[pl.ds(cc * CCAP + v * L, L)], t)
            return carry2

        lax.fori_loop(0, nocv, ov, jnp.int32(0))

    for rr in range(8):
        if rr + 1 < 8:
            rel_copy(rr + 1, True)
        rel_copy(rr, False).wait()

        def pair(j, carry, rr=rr):
            i0 = rr * 8 + 2 * j
            i1 = i0 + 1
            copies(i1, 1, True)
            wait_copies(i0, 0)
            compute(i0, 0, rr & 1)
            copies(jnp.minimum(i0 + 2, D - 2), 0, True)
            wait_copies(i1, 1)
            compute(i1, 1, rr & 1)
            return carry

        lax.fori_loop(0, 4, pair, jnp.int32(0))
    wait_copies(jnp.int32(D - 2), 0)

    NB = SLOTS // 128
    for k in range(NB):
        for t in range(128 // L):
            b2[k, pl.ds(t * L, L)] = s2_b[pl.ds(k * 128 + t * L, L)]
            b2[k + NB, pl.ds(t * L, L)] = o2_b[pl.ds(k * 128 + t * L, L)]
    swaits = []
    for k in range(NB):
        swaits.append(pltpu.async_copy(
            acc_s.at[pl.ds(k * 128, 128)], sh_s.at[b2.at[k]], sem,
            add=True))
        swaits.append(pltpu.async_copy(
            acc_ro.at[pl.ds(k * 128, 128)], sh_ro.at[b2.at[k + NB]], sem,
            add=True))
    for w in swaits:
        w.wait()
    plsc.subcore_barrier()
    sid = lax.axis_index("s")
    cid = lax.axis_index("c")

    @pl.when(sid < 8)
    def _writeback():
        off = sid * (B // 8)
        pltpu.sync_copy(sh_s.at[pl.ds(off, B // 8)],
                        out_s.at[cid, pl.ds(off, B // 8)])
        pltpu.sync_copy(sh_ro.at[pl.ds(off, B // 8)],
                        out_ro.at[cid, pl.ds(off, B // 8)])


def _mul_body(a0_ref, a1_ref, b0_ref, b1_ref, o_ref):
    o_ref[...] = ((a0_ref[...] + a1_ref[...])
                  * (b0_ref[...] + b1_ref[...]))


@jax.jit
def kernel(subjects, relations, objects, entity_table, relation_table):
    s = subjects.reshape(-1).astype(jnp.int32)
    r = relations.reshape(-1).astype(jnp.int32)
    o = objects.reshape(-1).astype(jnp.int32)
    ent_t = entity_table.T
    rel_flat = jnp.tile(jnp.pad(relation_table.T,
                                ((0, 0), (0, RELW - N_REL))).reshape(-1), NW // 8)
    tail_flat = jnp.pad(entity_table[TAIL0:].T,
                        ((0, 0), (0, 128 - NTAIL))).reshape(-1)
    mesh = plsc.VectorSubcoreMesh(core_axis_name="c", subcore_axis_name="s")
    run = functools.partial(
        pl.kernel,
        mesh=mesh,
        compiler_params=pltpu.CompilerParams(needs_layout_passes=False),
        out_type=(jax.ShapeDtypeStruct((NC, B), jnp.float32),
                  jax.ShapeDtypeStruct((NC, B), jnp.float32)),
        scratch_types=[
            pltpu.VMEM((CHUNK,), jnp.int32),
            pltpu.VMEM((CHUNK,), jnp.int32),
            pltpu.VMEM((CHUNK,), jnp.int32),
            pltpu.VMEM((CHUNK,), jnp.int32),
            pltpu.VMEM((CHUNK,), jnp.int32),
            pltpu.VMEM((CHUNK,), jnp.int32),
            pltpu.VMEM((CAP,), jnp.int32),
            pltpu.VMEM((CAP,), jnp.int32),
            pltpu.VMEM((CAP,), jnp.int32),
            pltpu.VMEM((CAP,), jnp.int32),
            pltpu.VMEM((CAP,), jnp.int32),
            pltpu.VMEM((SLOTS,), jnp.int32),
            pltpu.VMEM((SLOTS,), jnp.int32),
            pltpu.VMEM((SLOTS,), jnp.int32),
            pltpu.VMEM((SLOTS,), jnp.int32),
            pltpu.VMEM((SLOTS,), jnp.int32),
            pltpu.VMEM((2 * SLOTS // 128, 128), jnp.int32),
            pltpu.SMEM((2 * NCHUNK,), jnp.int32),
            pltpu.VMEM((8, BW), jnp.float32),
            pltpu.VMEM((8, BW), jnp.float32),
            pltpu.VMEM((8 * RELW,), jnp.float32),
            pltpu.VMEM((8 * RELW,), jnp.float32),
            pltpu.VMEM((D * 128,), jnp.float32),
            pltpu.VMEM((SLOTS,), jnp.float32),
            pltpu.VMEM((SLOTS,), jnp.float32),
            pltpu.VMEM((4096,), jnp.float32),
            pltpu.VMEM_SHARED((B + 64,), jnp.float32),
            pltpu.VMEM_SHARED((B + 64,), jnp.float32),
            pltpu.SemaphoreType.DMA,
            pltpu.SemaphoreType.DMA,
            pltpu.SemaphoreType.DMA,
            pltpu.SemaphoreType.DMA,
            pltpu.SemaphoreType.DMA,
            pltpu.SemaphoreType.DMA,
            pltpu.SemaphoreType.DMA,
        ],
    )(_score_body)
    out_s, out_ro = run(s, r, o, ent_t, rel_flat, tail_flat)
    scores = pl.pallas_call(
        _mul_body,
        out_shape=jax.ShapeDtypeStruct((128, 128), jnp.float32),
    )(out_s[0].reshape(128, 128), out_s[1].reshape(128, 128),
      out_ro[0].reshape(128, 128), out_ro[1].reshape(128, 128))
    return scores.reshape(B, 1)
